# Initial kernel scaffold; baseline (speedup 1.0000x reference)
import jax
import jax.numpy as jnp
from jax.experimental import pallas as pl


def kernel(prot_scalars, prot_vectors, lig_scalars, lig_vectors, pr_pr_eattr, lig_pr_eattr, pr_pr_edge_index, lig_pr_edge_index, lig_coords, backbone_coords, W_msg0, b_msg0, a_attn0, W_val0, W_head0, W_eup0, b_eup0, W_msg1, b_msg1, a_attn1, W_val1, W_head1, W_eup1, b_eup1, W_n1, b_n1, W_n2, b_n2, W_gate, W_flp, b_flp, W_fpp, b_fpp):
    raise NotImplementedError("write your pallas kernel here")


# speedup vs baseline: 1.9049x; 1.9049x over previous
import functools
import numpy as np
import jax
import jax.numpy as jnp
from jax import lax
from jax.experimental import pallas as pl
from jax.experimental.pallas import tpu as pltpu
from jax.experimental.pallas import tpu_sc as plsc

N_PR, N_LIG = 10000, 2000
E_PP, E_LP = 160000, 32000
NV, H = 4, 4

N_PAD = 10240
TRASH = 10000
EPP_PAD = 163840
ELP_PAD = 32768
NW = 32
CHUNK = 128


def _sel(shape, coords):
    m = np.zeros(shape, np.float32)
    for r, c in coords:
        m[r, c] = 1.0
    return m

_R_HEAD = _sel((128, 4), [(h * 32 + d, h) for h in range(4) for d in range(32)])
_P_HEAD = _sel((4, 128), [(h, h * 32 + d) for h in range(4) for d in range(32)])
_EYE4_16 = _sel((4, 16), [(i, i) for i in range(4)])
_SDEN_PP = _sel((128, 128), [(16 + h, h * 32 + d) for h in range(4) for d in range(32)])
_SDEN_LP = _sel((128, 128), [(20 + h, h * 32 + d) for h in range(4) for d in range(32)])
_P1X = _sel((128, 48), [(j * 3 + k, j * 12 + m * 3 + k) for j in range(4) for m in range(4) for k in range(3)])
_P2X = _sel((128, 48), [(m * 3 + k, j * 12 + m * 3 + k) for j in range(4) for m in range(4) for k in range(3)])
_R48 = _sel((48, 16), [(j * 12 + m * 3 + k, j * 4 + m) for j in range(4) for m in range(4) for k in range(3)])
_PL = _sel((128, 12), [(k, j * 3 + k) for j in range(4) for k in range(3)])
_PB = _sel((128, 12), [(24 + k, j * 3 + k) for j in range(4) for k in range(3)])
_PNV = _sel((128, 12), [(i, i) for i in range(12)])
_R12 = _sel((12, 4), [(j * 3 + k, j) for j in range(4) for k in range(3)])
_C12 = np.full((12, 1), 0.25, np.float32)
_G16 = _sel((16, 16), [(j, j * 3 + k) for j in range(4) for k in range(3)])
_N16 = _sel((16, 16), [(j * 3 + k, j) for j in range(4) for k in range(3)])
_K1 = _sel((16, 128), [(i, i) for i in range(12)])
_S4PP = _sel((4, 128), [(h, 16 + h) for h in range(4)])
_S4LP = _sel((4, 128), [(h, 20 + h) for h in range(4)])
_KB = _sel((16, 128), [(k, 24 + k) for k in range(3)])


def _gather(table, idx, width):
    etot = idx.shape[0]
    cpt = etot // (NW * CHUNK)
    mesh = plsc.VectorSubcoreMesh(core_axis_name="c", subcore_axis_name="s",
                                  num_cores=2)

    @functools.partial(
        pl.kernel, mesh=mesh,
        out_type=jax.ShapeDtypeStruct((etot, width), jnp.float32),
        scratch_types=[
            pltpu.VMEM((CHUNK,), jnp.int32),
            pltpu.VMEM((CHUNK, width), jnp.float32),
            pltpu.SemaphoreType.DMA,
        ],
    )
    def k(tab_hbm, idx_hbm, out_hbm, idx_v, rows_v, sem):
        wid = lax.axis_index("c") * 16 + lax.axis_index("s")
        base = wid * (cpt * CHUNK)

        def body(j, _):
            off = base + j * CHUNK
            pltpu.sync_copy(idx_hbm.at[pl.ds(off, CHUNK)], idx_v)
            pltpu.async_copy(tab_hbm.at[idx_v], rows_v, sem).wait()
            pltpu.sync_copy(rows_v, out_hbm.at[pl.ds(off, CHUNK)])
            return _

        lax.fori_loop(0, cpt, body, 0)

    return k(table, idx)


def _scatter(ev_pp, ex_pp, idx_pp, idx4_pp, ev_lp, ex_lp, idx_lp, idx4_lp,
             zeros, zeros_den):
    cpt_pp = idx_pp.shape[0] // NW
    cpt_lp = idx_lp.shape[0] // NW
    zrows = N_PAD // 16
    dlen = (N_PAD * 4) // 16
    mesh = plsc.VectorSubcoreMesh(core_axis_name="c", subcore_axis_name="s",
                                  num_cores=2)

    @functools.partial(
        pl.kernel, mesh=mesh,
        out_type=[jax.ShapeDtypeStruct((2 * N_PAD, 128), jnp.float32),
                  jax.ShapeDtypeStruct((2, N_PAD * 4), jnp.float32),
                  jax.ShapeDtypeStruct((2 * N_PAD, 128), jnp.float32),
                  jax.ShapeDtypeStruct((2, N_PAD * 4), jnp.float32)],
        scratch_types=[
            pltpu.VMEM((cpt_pp, CHUNK), jnp.int32),
            pltpu.VMEM((4, CHUNK), jnp.int32),
            pltpu.VMEM((4, CHUNK), jnp.float32),
            pltpu.VMEM((CHUNK, 128), jnp.float32),
            pltpu.VMEM_SHARED((N_PAD, 128), jnp.float32),
            pltpu.VMEM_SHARED((N_PAD * 4,), jnp.float32),
        ],
    )
    def k(evp_hbm, exp_hbm, ip_hbm, i4p_hbm, evl_hbm, exl_hbm, il_hbm, i4l_hbm,
          z_hbm, zd_hbm, out_pp, outd_pp, out_lp, outd_lp,
          idx_v, idx4_v, ex_v, ev_v, acc, accd):
        c = lax.axis_index("c")
        s = lax.axis_index("s")

        def phase(ev_hbm, ex_hbm, idx_hbm, idx4_hbm, out_hbm, outd_hbm, cpt, half):
            pltpu.sync_copy(z_hbm.at[pl.ds(s * zrows, zrows)],
                            acc.at[pl.ds(s * zrows, zrows)])
            pltpu.sync_copy(zd_hbm.at[pl.ds(s * dlen, dlen)],
                            accd.at[pl.ds(s * dlen, dlen)])
            plsc.subcore_barrier()
            rowbase = c * half + s * cpt
            pltpu.sync_copy(idx_hbm.at[pl.ds(rowbase, cpt)],
                            idx_v.at[pl.ds(0, cpt)])

            def body(j, _):
                pltpu.sync_copy(ev_hbm.at[pl.ds((rowbase + j) * CHUNK, CHUNK)],
                                ev_v)
                pltpu.sync_copy(idx4_hbm.at[pl.ds((rowbase + j) * 4, 4)], idx4_v)
                pltpu.sync_copy(ex_hbm.at[pl.ds((rowbase + j) * 4, 4)], ex_v)
                pltpu.sync_copy(ev_v, acc.at[idx_v.at[j]], add=True)
                for r in range(4):
                    pltpu.sync_copy(ex_v.at[r], accd.at[idx4_v.at[r]], add=True)
                return _

            lax.fori_loop(0, cpt, body, 0)
            plsc.subcore_barrier()
            pltpu.sync_copy(acc.at[pl.ds(s * zrows, zrows)],
                            out_hbm.at[pl.ds(c * N_PAD + s * zrows, zrows)])
            pltpu.sync_copy(accd.at[pl.ds(s * dlen, dlen)],
                            outd_hbm.at[c].at[pl.ds(s * dlen, dlen)])
            plsc.subcore_barrier()

        phase(evp_hbm, exp_hbm, ip_hbm, i4p_hbm, out_pp, outd_pp,
              cpt_pp, cpt_pp * 16)
        phase(evl_hbm, exl_hbm, il_hbm, i4l_hbm, out_lp, outd_lp,
              cpt_lp, cpt_lp * 16)

    return k(ev_pp, ex_pp, idx_pp, idx4_pp, ev_lp, ex_lp, idx_lp, idx4_lp,
             zeros, zeros_den)


_BLK_E = 2048
_BLK_N = 512


def _full(shape):
    return pl.BlockSpec(shape, lambda i: (0, 0))


def _rows(shape):
    return pl.BlockSpec(shape, lambda i: (i, 0))


def _edge_a_body(src_ref, dst_ref, ea_ref, w1, w2, w3, v1, v2, v3, bm, af,
                 rh, ph, out_ref, ex_ref):
    src, dst, ea = src_ref[...], dst_ref[...], ea_ref[...]
    pre = src @ w1[...] + dst @ w2[...] + ea @ w3[...] + bm[...]
    h = jnp.where(pre >= 0, pre, 0.2 * pre)
    ex4 = jnp.exp((h * af[...]) @ rh[...])
    vals = src @ v1[...] + dst @ v2[...] + ea @ v3[...]
    out_ref[...] = vals * (ex4 @ ph[...])
    ex_ref[...] = ex4


def _edge_a(srcg, dstg, eattr, W_msg, b_msg, a_attn, W_val):
    epad = srcg.shape[0]
    w = [W_msg[:128], W_msg[128:256], W_msg[256:384],
         W_val[:128], W_val[128:256], W_val[256:384]]
    consts = [b_msg.reshape(1, 128), a_attn.reshape(1, 128),
              jnp.asarray(_R_HEAD), jnp.asarray(_P_HEAD)]
    grid = epad // _BLK_E
    return pl.pallas_call(
        _edge_a_body,
        grid=(grid,),
        in_specs=[_rows((_BLK_E, 128))] * 3
        + [_full((128, 128))] * 6
        + [_full((1, 128)), _full((1, 128)), _full((128, 4)), _full((4, 128))],
        out_specs=[_rows((_BLK_E, 128)), _rows((_BLK_E, 4))],
        out_shape=[jax.ShapeDtypeStruct((epad, 128), jnp.float32),
                   jax.ShapeDtypeStruct((epad, 4), jnp.float32)],
    )(srcg, dstg, eattr, *w, *consts)


def _node_body(prot_ref, pvec_ref, bb_ref, npp0, npp1, nlp0, nlp1,
               dpp0, dpp1, dlp0, dlp1,
               a1, a2, a3, bn1, wn2, bn2, wg, wh0, wh1,
               ph, g16, n16, k1, s4pp, s4lp, kb,
               ns_ref, nvec_ref, geom_ref):
    prot = prot_ref[...]
    dpp = dpp0[...] + dpp1[...]
    dlp = dlp0[...] + dlp1[...]
    agg0 = (npp0[...] + npp1[...]) / (dpp @ ph[...] + 1e-9)
    agg1 = (nlp0[...] + nlp1[...]) / (dlp @ ph[...] + 1e-9)
    u = jnp.maximum(prot @ a1[...] + (agg0 @ wh0[...]) @ a2[...]
                    + (agg1 @ wh1[...]) @ a3[...] + bn1[...], 0.0)
    new_scal = prot + u @ wn2[...] + bn2[...]
    g = jax.nn.sigmoid(new_scal @ wg[...])
    nvec = pvec_ref[...] * (g @ g16[...])
    n2 = (nvec * nvec) @ n16[...]
    rn = 1.0 / jnp.sqrt(jnp.maximum(n2, 1e-8))
    nv = nvec * (rn @ g16[...])
    ns_ref[...] = new_scal
    nvec_ref[...] = nvec
    geom_ref[...] = (nv @ k1[...] + dpp @ s4pp[...] + dlp @ s4lp[...]
                     + bb_ref[...] @ kb[...])


def _node(prot, pvec16, bb16, npp0, npp1, nlp0, nlp1, dpp0, dpp1, dlp0, dlp1,
          W_n1, b_n1, W_n2, b_n2, Wg16, Wh0, Wh1):
    grid = N_PAD // _BLK_N
    consts = [jnp.asarray(m) for m in
              (_P_HEAD, _G16, _N16, _K1, _S4PP, _S4LP, _KB)]
    return pl.pallas_call(
        _node_body,
        grid=(grid,),
        in_specs=[_rows((_BLK_N, 128)), _rows((_BLK_N, 16)), _rows((_BLK_N, 16))]
        + [_rows((_BLK_N, 128))] * 4
        + [_rows((_BLK_N, 4))] * 4
        + [_full((128, 128))] * 3
        + [_full((1, 128)), _full((128, 128)), _full((1, 128)),
           _full((128, 16)), _full((128, 128)), _full((128, 128))]
        + [_full((4, 128)), _full((16, 16)), _full((16, 16)),
           _full((16, 128)), _full((4, 128)), _full((4, 128)), _full((16, 128))],
        out_specs=[_rows((_BLK_N, 128)), _rows((_BLK_N, 16)), _rows((_BLK_N, 128))],
        out_shape=[jax.ShapeDtypeStruct((N_PAD, 128), jnp.float32),
                   jax.ShapeDtypeStruct((N_PAD, 16), jnp.float32),
                   jax.ShapeDtypeStruct((N_PAD, 128), jnp.float32)],
    )(prot, pvec16, bb16, npp0, npp1, nlp0, nlp1, dpp0, dpp1, dlp0, dlp1,
      W_n1[:128], W_n1[128:256], W_n1[256:384],
      b_n1.reshape(1, 128), W_n2, b_n2.reshape(1, 128),
      Wg16, Wh0, Wh1, *consts)


def _edge_b_pp_body(ev_ref, ea_ref, ts_ref, td_ref, e1, e2, be, wf1, wf2, bf,
                    sden, p1x, p2x, r48, out_ref):
    ev, ea, ts, td = ev_ref[...], ea_ref[...], ts_ref[...], td_ref[...]
    w = ev / (td @ sden[...] + 1e-9)
    new_pp = jnp.maximum(ea @ e1[...] + w @ e2[...] + be[...], 0.0)
    t48 = (ts @ p1x[...]) * (td @ p2x[...])
    out_ref[...] = new_pp @ wf1[...] + (t48 @ r48[...]) @ wf2[...] + bf[...]


def _edge_b_pp(ev, eattr, tsrc, tdst, W_eup, b_eup, W_fpp, b_fpp):
    epad = ev.shape[0]
    consts = [jnp.asarray(m) for m in (_SDEN_PP, _P1X, _P2X, _R48)]
    return pl.pallas_call(
        _edge_b_pp_body,
        grid=(epad // _BLK_E,),
        in_specs=[_rows((_BLK_E, 128)), _rows((_BLK_E, 128)),
                  _rows((_BLK_E, 128)), _rows((_BLK_E, 128)),
                  _full((128, 128)), _full((128, 128)), _full((1, 128)),
                  _full((128, 128)), _full((16, 128)), _full((1, 128)),
                  _full((128, 128)), _full((128, 48)), _full((128, 48)),
                  _full((48, 16))],
        out_specs=_rows((_BLK_E, 128)),
        out_shape=jax.ShapeDtypeStruct((epad, 128), jnp.float32),
    )(ev, eattr, tsrc, tdst, W_eup[:128], W_eup[128:], b_eup.reshape(1, 128),
      W_fpp[:128], W_fpp[128:], b_fpp.reshape(1, 128), *consts)


def _edge_b_lp_body(ev_ref, ea_ref, td_ref, tl_ref, e1, e2, be, wf1, wf2, bf,
                    sden, plm, pbm, pnv, r12, c12, out_ref):
    ev, ea, td, tl = ev_ref[...], ea_ref[...], td_ref[...], tl_ref[...]
    w = ev / (td @ sden[...] + 1e-9)
    new_lp = jnp.maximum(ea @ e1[...] + w @ e2[...] + be[...], 0.0)
    d12 = tl @ plm[...] - td @ pbm[...]
    n2 = (d12 * d12) @ c12[...]
    rn = 1.0 / jnp.sqrt(jnp.maximum(n2, 1e-8))
    t12 = (td @ pnv[...]) * (d12 * rn)
    out_ref[...] = new_lp @ wf1[...] + (t12 @ r12[...]) @ wf2[...] + bf[...]


def _edge_b_lp(ev, eattr, tdst, tlig, W_eup, b_eup, W_flp, b_flp):
    epad = ev.shape[0]
    consts = [jnp.asarray(m) for m in (_SDEN_LP, _PL, _PB, _PNV, _R12, _C12)]
    return pl.pallas_call(
        _edge_b_lp_body,
        grid=(epad // _BLK_E,),
        in_specs=[_rows((_BLK_E, 128)), _rows((_BLK_E, 128)),
                  _rows((_BLK_E, 128)), _rows((_BLK_E, 128)),
                  _full((128, 128)), _full((128, 128)), _full((1, 128)),
                  _full((128, 128)), _full((4, 128)), _full((1, 128)),
                  _full((128, 128)), _full((128, 12)), _full((128, 12)),
                  _full((128, 12)), _full((12, 4)), _full((12, 1))],
        out_specs=_rows((_BLK_E, 128)),
        out_shape=jax.ShapeDtypeStruct((epad, 128), jnp.float32),
    )(ev, eattr, tdst, tlig, W_eup[:128], W_eup[128:], b_eup.reshape(1, 128),
      W_flp[:128], W_flp[128:], b_flp.reshape(1, 128), *consts)


def _pad_rows(x, n):
    return jnp.pad(x, ((0, n - x.shape[0]),) + ((0, 0),) * (x.ndim - 1))


@jax.jit
def _run(prot_scalars, prot_vectors, lig_scalars, lig_vectors, pr_pr_eattr,
         lig_pr_eattr, pr_pr_edge_index, lig_pr_edge_index, lig_coords,
         backbone_coords, W_msg0, b_msg0, a_attn0, W_val0, W_head0, W_eup0,
         b_eup0, W_msg1, b_msg1, a_attn1, W_val1, W_head1, W_eup1, b_eup1,
         W_n1, b_n1, W_n2, b_n2, W_gate, W_flp, b_flp, W_fpp, b_fpp):
    prot = _pad_rows(prot_scalars, N_PAD)
    tab1 = jnp.concatenate([prot, lig_scalars], axis=0)
    ep, el = pr_pr_edge_index, lig_pr_edge_index
    pps = jnp.pad(ep[0], (0, EPP_PAD - E_PP))
    ppd = jnp.pad(ep[1], (0, EPP_PAD - E_PP))
    lps = jnp.pad(el[0], (0, ELP_PAD - E_LP))
    lpd = jnp.pad(el[1], (0, ELP_PAD - E_LP))
    ppd_sc = jnp.pad(ep[1], (0, EPP_PAD - E_PP), constant_values=TRASH)
    lpd_sc = jnp.pad(el[1], (0, ELP_PAD - E_LP), constant_values=TRASH)

    idx1 = jnp.concatenate([pps, ppd, lps + N_PAD, lpd])
    g1 = _gather(tab1, idx1, 128)
    pp_src_g = g1[:EPP_PAD]
    pp_dst_g = g1[EPP_PAD:2 * EPP_PAD]
    lp_src_g = g1[2 * EPP_PAD:2 * EPP_PAD + ELP_PAD]
    lp_dst_g = g1[2 * EPP_PAD + ELP_PAD:]

    eattr_pp = _pad_rows(pr_pr_eattr, EPP_PAD)
    eattr_lp = _pad_rows(lig_pr_eattr, ELP_PAD)

    ev_pp, ex_pp = _edge_a(pp_src_g, pp_dst_g, eattr_pp, W_msg0, b_msg0, a_attn0, W_val0)
    ev_lp, ex_lp = _edge_a(lp_src_g, lp_dst_g, eattr_lp, W_msg1, b_msg1, a_attn1, W_val1)

    zeros = jnp.zeros((N_PAD, 128), jnp.float32)
    zeros_den = jnp.zeros((N_PAD * 4,), jnp.float32)
    idx4_pp = (ppd_sc[:, None] * 4 + jnp.arange(4, dtype=jnp.int32)[None, :])
    idx4_lp = (lpd_sc[:, None] * 4 + jnp.arange(4, dtype=jnp.int32)[None, :])
    npp, dpp, nlp, dlp = _scatter(
        ev_pp, ex_pp.reshape(-1, CHUNK),
        ppd_sc.reshape(-1, CHUNK), idx4_pp.reshape(-1, CHUNK),
        ev_lp, ex_lp.reshape(-1, CHUNK),
        lpd_sc.reshape(-1, CHUNK), idx4_lp.reshape(-1, CHUNK),
        zeros, zeros_den)
    dpp = dpp.reshape(2, N_PAD, 4)
    dlp = dlp.reshape(2, N_PAD, 4)

    pvec16 = jnp.pad(_pad_rows(prot_vectors.reshape(N_PR, 12), N_PAD),
                     ((0, 0), (0, 4)))
    bb16 = jnp.pad(_pad_rows(backbone_coords[:, 1], N_PAD), ((0, 0), (0, 13)))
    Wg16 = jnp.pad(W_gate, ((0, 0), (0, 12)))
    new_scal, nvec16, geom = _node(
        prot, pvec16, bb16, npp[:N_PAD], npp[N_PAD:], nlp[:N_PAD], nlp[N_PAD:],
        dpp[0], dpp[1], dlp[0], dlp[1],
        W_n1, b_n1, W_n2, b_n2, Wg16, W_head0, W_head1)

    ligtab = jnp.pad(lig_coords, ((0, 0), (0, 125)))
    tab2 = jnp.concatenate([geom, ligtab], axis=0)
    idx2 = jnp.concatenate([pps, ppd, lpd, lps + N_PAD])
    g2 = _gather(tab2, idx2, 128)
    tsrc_pp = g2[:EPP_PAD]
    tdst_pp = g2[EPP_PAD:2 * EPP_PAD]
    tdst_lp = g2[2 * EPP_PAD:2 * EPP_PAD + ELP_PAD]
    tlig = g2[2 * EPP_PAD + ELP_PAD:]

    pp_out = _edge_b_pp(ev_pp, eattr_pp, tsrc_pp, tdst_pp,
                        W_eup0, b_eup0, W_fpp, b_fpp)
    lp_out = _edge_b_lp(ev_lp, eattr_lp, tdst_lp, tlig,
                        W_eup1, b_eup1, W_flp, b_flp)

    return (new_scal[:N_PR],
            nvec16[:N_PR, :12].reshape(N_PR, NV, 3),
            pp_out[:E_PP],
            lp_out[:E_LP])


def kernel(*args):
    return _run(*args)


# speedup vs baseline: 2.0894x; 1.0969x over previous
import functools
import numpy as np
import jax
import jax.numpy as jnp
from jax import lax
from jax.experimental import pallas as pl
from jax.experimental.pallas import tpu as pltpu
from jax.experimental.pallas import tpu_sc as plsc

N_PR, N_LIG = 10000, 2000
E_PP, E_LP = 160000, 32000
NV, H = 4, 4

N_PAD = 10240
TRASH = 10000
EPP_PAD = 163840
ELP_PAD = 32768
NW = 32
CHUNK = 128


def _sel(shape, coords):
    m = np.zeros(shape, np.float32)
    for r, c in coords:
        m[r, c] = 1.0
    return m

_R_HEAD = _sel((128, 4), [(h * 32 + d, h) for h in range(4) for d in range(32)])
_P_HEAD = _sel((4, 128), [(h, h * 32 + d) for h in range(4) for d in range(32)])
_EYE4_16 = _sel((4, 16), [(i, i) for i in range(4)])
_SDEN_PP = _sel((128, 128), [(16 + h, h * 32 + d) for h in range(4) for d in range(32)])
_SDEN_LP = _sel((128, 128), [(20 + h, h * 32 + d) for h in range(4) for d in range(32)])
_P1X = _sel((128, 48), [(j * 3 + k, j * 12 + m * 3 + k) for j in range(4) for m in range(4) for k in range(3)])
_P2X = _sel((128, 48), [(m * 3 + k, j * 12 + m * 3 + k) for j in range(4) for m in range(4) for k in range(3)])
_R48 = _sel((48, 16), [(j * 12 + m * 3 + k, j * 4 + m) for j in range(4) for m in range(4) for k in range(3)])
_PL = _sel((128, 12), [(k, j * 3 + k) for j in range(4) for k in range(3)])
_PB = _sel((128, 12), [(24 + k, j * 3 + k) for j in range(4) for k in range(3)])
_PNV = _sel((128, 12), [(i, i) for i in range(12)])
_R12 = _sel((12, 4), [(j * 3 + k, j) for j in range(4) for k in range(3)])
_C12 = np.full((12, 1), 0.25, np.float32)
_G16 = _sel((16, 16), [(j, j * 3 + k) for j in range(4) for k in range(3)])
_N16 = _sel((16, 16), [(j * 3 + k, j) for j in range(4) for k in range(3)])
_K1 = _sel((16, 128), [(i, i) for i in range(12)])
_S4PP = _sel((4, 128), [(h, 16 + h) for h in range(4)])
_S4LP = _sel((4, 128), [(h, 20 + h) for h in range(4)])
_KB = _sel((16, 128), [(k, 24 + k) for k in range(3)])


def _gather(table, idx, width):
    etot = idx.shape[0]
    cpt = etot // (NW * CHUNK)
    mesh = plsc.VectorSubcoreMesh(core_axis_name="c", subcore_axis_name="s",
                                  num_cores=2)

    @functools.partial(
        pl.kernel, mesh=mesh,
        out_type=jax.ShapeDtypeStruct((etot, width), jnp.float32),
        scratch_types=[
            pltpu.VMEM((2, CHUNK), jnp.int32),
            pltpu.VMEM((2, CHUNK, width), jnp.float32),
            pltpu.SemaphoreType.DMA,
            pltpu.SemaphoreType.DMA,
            pltpu.SemaphoreType.DMA,
        ],
    )
    def k(tab_hbm, idx_hbm, out_hbm, idx_v, rows_v, sidx, sgat, sout):
        wid = lax.axis_index("c") * 16 + lax.axis_index("s")
        base = wid * (cpt * CHUNK)
        pltpu.sync_copy(idx_hbm.at[pl.ds(base, CHUNK)], idx_v.at[0])

        def body(j, _):
            slot = j % 2

            @pl.when(j >= 2)
            def _drain_out():
                pltpu.make_async_copy(
                    rows_v.at[slot], out_hbm.at[pl.ds(base, CHUNK)], sout
                ).wait()

            @pl.when(j >= 1)
            def _wait_idx():
                pltpu.make_async_copy(
                    idx_hbm.at[pl.ds(base, CHUNK)], idx_v.at[slot], sidx
                ).wait()

            gat = pltpu.async_copy(tab_hbm.at[idx_v.at[slot]],
                                   rows_v.at[slot], sgat)

            @pl.when(j + 1 < cpt)
            def _next_idx():
                pltpu.async_copy(
                    idx_hbm.at[pl.ds(base + (j + 1) * CHUNK, CHUNK)],
                    idx_v.at[(j + 1) % 2], sidx)

            gat.wait()
            pltpu.async_copy(rows_v.at[slot],
                             out_hbm.at[pl.ds(base + j * CHUNK, CHUNK)], sout)
            return _

        lax.fori_loop(0, cpt, body, 0)
        for _ in range(2):
            pltpu.make_async_copy(
                rows_v.at[0], out_hbm.at[pl.ds(base, CHUNK)], sout).wait()

    return k(table, idx)


def _scatter(ev_pp, ex_pp, idx_pp, idx4_pp, ev_lp, ex_lp, idx_lp, idx4_lp,
             zeros, zeros_den):
    cpt_pp = idx_pp.shape[0] // NW
    cpt_lp = idx_lp.shape[0] // NW
    zrows = N_PAD // 16
    dlen = (N_PAD * 4) // 16
    mesh = plsc.VectorSubcoreMesh(core_axis_name="c", subcore_axis_name="s",
                                  num_cores=2)

    @functools.partial(
        pl.kernel, mesh=mesh,
        out_type=[jax.ShapeDtypeStruct((2 * N_PAD, 128), jnp.float32),
                  jax.ShapeDtypeStruct((2, N_PAD * 4), jnp.float32),
                  jax.ShapeDtypeStruct((2 * N_PAD, 128), jnp.float32),
                  jax.ShapeDtypeStruct((2, N_PAD * 4), jnp.float32)],
        scratch_types=[
            pltpu.VMEM((8, CHUNK), jnp.int32),
            pltpu.VMEM((32, CHUNK), jnp.int32),
            pltpu.VMEM((32, CHUNK), jnp.float32),
            pltpu.VMEM((2, CHUNK, 128), jnp.float32),
            pltpu.SemaphoreType.DMA,
            pltpu.VMEM_SHARED((N_PAD, 128), jnp.float32),
            pltpu.VMEM_SHARED((N_PAD * 4,), jnp.float32),
        ],
    )
    def k(evp_hbm, exp_hbm, ip_hbm, i4p_hbm, evl_hbm, exl_hbm, il_hbm, i4l_hbm,
          z_hbm, zd_hbm, out_pp, outd_pp, out_lp, outd_lp,
          idx_v, idx4_v, ex_v, ev_v, sev, acc, accd):
        c = lax.axis_index("c")
        s = lax.axis_index("s")
        GR = 8

        def phase(ev_hbm, ex_hbm, idx_hbm, idx4_hbm, out_hbm, outd_hbm, cpt, half):
            pltpu.sync_copy(z_hbm.at[pl.ds(s * zrows, zrows)],
                            acc.at[pl.ds(s * zrows, zrows)])
            pltpu.sync_copy(zd_hbm.at[pl.ds(s * dlen, dlen)],
                            accd.at[pl.ds(s * dlen, dlen)])
            plsc.subcore_barrier()
            rowbase = c * half + s * cpt
            pltpu.sync_copy(ev_hbm.at[pl.ds(rowbase * CHUNK, CHUNK)],
                            ev_v.at[0])

            def group(g, _):
                base = rowbase + g * GR
                pltpu.sync_copy(idx_hbm.at[pl.ds(base, GR)], idx_v)
                pltpu.sync_copy(idx4_hbm.at[pl.ds(base * 4, GR * 4)], idx4_v)
                pltpu.sync_copy(ex_hbm.at[pl.ds(base * 4, GR * 4)], ex_v)
                for t in range(GR):
                    j = g * GR + t
                    slot = t % 2

                    @pl.when(j >= 1)
                    def _wait_ev():
                        pltpu.make_async_copy(
                            ev_hbm.at[pl.ds(rowbase * CHUNK, CHUNK)],
                            ev_v.at[slot], sev).wait()

                    @pl.when(j + 1 < cpt)
                    def _next_ev():
                        pltpu.async_copy(
                            ev_hbm.at[pl.ds((base + t + 1) * CHUNK, CHUNK)],
                            ev_v.at[(t + 1) % 2], sev)

                    pltpu.sync_copy(ev_v.at[slot], acc.at[idx_v.at[t]],
                                    add=True)
                    for r in range(4):
                        pltpu.sync_copy(ex_v.at[t * 4 + r],
                                        accd.at[idx4_v.at[t * 4 + r]],
                                        add=True)
                return _

            lax.fori_loop(0, cpt // GR, group, 0)
            plsc.subcore_barrier()
            pltpu.sync_copy(acc.at[pl.ds(s * zrows, zrows)],
                            out_hbm.at[pl.ds(c * N_PAD + s * zrows, zrows)])
            pltpu.sync_copy(accd.at[pl.ds(s * dlen, dlen)],
                            outd_hbm.at[c].at[pl.ds(s * dlen, dlen)])
            plsc.subcore_barrier()

        phase(evp_hbm, exp_hbm, ip_hbm, i4p_hbm, out_pp, outd_pp,
              cpt_pp, cpt_pp * 16)
        phase(evl_hbm, exl_hbm, il_hbm, i4l_hbm, out_lp, outd_lp,
              cpt_lp, cpt_lp * 16)

    return k(ev_pp, ex_pp, idx_pp, idx4_pp, ev_lp, ex_lp, idx_lp, idx4_lp,
             zeros, zeros_den)


_BLK_E = 2048
_BLK_N = 512


def _full(shape):
    return pl.BlockSpec(shape, lambda i: (0, 0))


def _rows(shape):
    return pl.BlockSpec(shape, lambda i: (i, 0))


def _edge_a_body(src_ref, dst_ref, ea_ref, w1, w2, w3, v1, v2, v3, bm, af,
                 rh, ph, out_ref, ex_ref):
    src, dst, ea = src_ref[...], dst_ref[...], ea_ref[...]
    pre = src @ w1[...] + dst @ w2[...] + ea @ w3[...] + bm[...]
    h = jnp.where(pre >= 0, pre, 0.2 * pre)
    ex4 = jnp.exp((h * af[...]) @ rh[...])
    vals = src @ v1[...] + dst @ v2[...] + ea @ v3[...]
    out_ref[...] = vals * (ex4 @ ph[...])
    ex_ref[...] = ex4


def _edge_a(srcg, dstg, eattr, W_msg, b_msg, a_attn, W_val):
    epad = srcg.shape[0]
    w = [W_msg[:128], W_msg[128:256], W_msg[256:384],
         W_val[:128], W_val[128:256], W_val[256:384]]
    consts = [b_msg.reshape(1, 128), a_attn.reshape(1, 128),
              jnp.asarray(_R_HEAD), jnp.asarray(_P_HEAD)]
    grid = epad // _BLK_E
    return pl.pallas_call(
        _edge_a_body,
        grid=(grid,),
        in_specs=[_rows((_BLK_E, 128))] * 3
        + [_full((128, 128))] * 6
        + [_full((1, 128)), _full((1, 128)), _full((128, 4)), _full((4, 128))],
        out_specs=[_rows((_BLK_E, 128)), _rows((_BLK_E, 4))],
        out_shape=[jax.ShapeDtypeStruct((epad, 128), jnp.float32),
                   jax.ShapeDtypeStruct((epad, 4), jnp.float32)],
    )(srcg, dstg, eattr, *w, *consts)


def _node_body(prot_ref, pvec_ref, bb_ref, npp0, npp1, nlp0, nlp1,
               dpp0, dpp1, dlp0, dlp1,
               a1, a2, a3, bn1, wn2, bn2, wg, wh0, wh1,
               ph, g16, n16, k1, s4pp, s4lp, kb,
               ns_ref, nvec_ref, geom_ref):
    prot = prot_ref[...]
    dpp = dpp0[...] + dpp1[...]
    dlp = dlp0[...] + dlp1[...]
    agg0 = (npp0[...] + npp1[...]) / (dpp @ ph[...] + 1e-9)
    agg1 = (nlp0[...] + nlp1[...]) / (dlp @ ph[...] + 1e-9)
    u = jnp.maximum(prot @ a1[...] + (agg0 @ wh0[...]) @ a2[...]
                    + (agg1 @ wh1[...]) @ a3[...] + bn1[...], 0.0)
    new_scal = prot + u @ wn2[...] + bn2[...]
    g = jax.nn.sigmoid(new_scal @ wg[...])
    nvec = pvec_ref[...] * (g @ g16[...])
    n2 = (nvec * nvec) @ n16[...]
    rn = 1.0 / jnp.sqrt(jnp.maximum(n2, 1e-8))
    nv = nvec * (rn @ g16[...])
    ns_ref[...] = new_scal
    nvec_ref[...] = nvec
    geom_ref[...] = (nv @ k1[...] + dpp @ s4pp[...] + dlp @ s4lp[...]
                     + bb_ref[...] @ kb[...])


def _node(prot, pvec16, bb16, npp0, npp1, nlp0, nlp1, dpp0, dpp1, dlp0, dlp1,
          W_n1, b_n1, W_n2, b_n2, Wg16, Wh0, Wh1):
    grid = N_PAD // _BLK_N
    consts = [jnp.asarray(m) for m in
              (_P_HEAD, _G16, _N16, _K1, _S4PP, _S4LP, _KB)]
    return pl.pallas_call(
        _node_body,
        grid=(grid,),
        in_specs=[_rows((_BLK_N, 128)), _rows((_BLK_N, 16)), _rows((_BLK_N, 16))]
        + [_rows((_BLK_N, 128))] * 4
        + [_rows((_BLK_N, 4))] * 4
        + [_full((128, 128))] * 3
        + [_full((1, 128)), _full((128, 128)), _full((1, 128)),
           _full((128, 16)), _full((128, 128)), _full((128, 128))]
        + [_full((4, 128)), _full((16, 16)), _full((16, 16)),
           _full((16, 128)), _full((4, 128)), _full((4, 128)), _full((16, 128))],
        out_specs=[_rows((_BLK_N, 128)), _rows((_BLK_N, 16)), _rows((_BLK_N, 128))],
        out_shape=[jax.ShapeDtypeStruct((N_PAD, 128), jnp.float32),
                   jax.ShapeDtypeStruct((N_PAD, 16), jnp.float32),
                   jax.ShapeDtypeStruct((N_PAD, 128), jnp.float32)],
    )(prot, pvec16, bb16, npp0, npp1, nlp0, nlp1, dpp0, dpp1, dlp0, dlp1,
      W_n1[:128], W_n1[128:256], W_n1[256:384],
      b_n1.reshape(1, 128), W_n2, b_n2.reshape(1, 128),
      Wg16, Wh0, Wh1, *consts)


def _edge_b_pp_body(ev_ref, ea_ref, ts_ref, td_ref, e1, e2, be, wf1, wf2, bf,
                    sden, p1x, p2x, r48, out_ref):
    ev, ea, ts, td = ev_ref[...], ea_ref[...], ts_ref[...], td_ref[...]
    w = ev / (td @ sden[...] + 1e-9)
    new_pp = jnp.maximum(ea @ e1[...] + w @ e2[...] + be[...], 0.0)
    t48 = (ts @ p1x[...]) * (td @ p2x[...])
    out_ref[...] = new_pp @ wf1[...] + (t48 @ r48[...]) @ wf2[...] + bf[...]


def _edge_b_pp(ev, eattr, tsrc, tdst, W_eup, b_eup, W_fpp, b_fpp):
    epad = ev.shape[0]
    consts = [jnp.asarray(m) for m in (_SDEN_PP, _P1X, _P2X, _R48)]
    return pl.pallas_call(
        _edge_b_pp_body,
        grid=(epad // _BLK_E,),
        in_specs=[_rows((_BLK_E, 128)), _rows((_BLK_E, 128)),
                  _rows((_BLK_E, 128)), _rows((_BLK_E, 128)),
                  _full((128, 128)), _full((128, 128)), _full((1, 128)),
                  _full((128, 128)), _full((16, 128)), _full((1, 128)),
                  _full((128, 128)), _full((128, 48)), _full((128, 48)),
                  _full((48, 16))],
        out_specs=_rows((_BLK_E, 128)),
        out_shape=jax.ShapeDtypeStruct((epad, 128), jnp.float32),
    )(ev, eattr, tsrc, tdst, W_eup[:128], W_eup[128:], b_eup.reshape(1, 128),
      W_fpp[:128], W_fpp[128:], b_fpp.reshape(1, 128), *consts)


def _edge_b_lp_body(ev_ref, ea_ref, td_ref, tl_ref, e1, e2, be, wf1, wf2, bf,
                    sden, plm, pbm, pnv, r12, c12, out_ref):
    ev, ea, td, tl = ev_ref[...], ea_ref[...], td_ref[...], tl_ref[...]
    w = ev / (td @ sden[...] + 1e-9)
    new_lp = jnp.maximum(ea @ e1[...] + w @ e2[...] + be[...], 0.0)
    d12 = tl @ plm[...] - td @ pbm[...]
    n2 = (d12 * d12) @ c12[...]
    rn = 1.0 / jnp.sqrt(jnp.maximum(n2, 1e-8))
    t12 = (td @ pnv[...]) * (d12 * rn)
    out_ref[...] = new_lp @ wf1[...] + (t12 @ r12[...]) @ wf2[...] + bf[...]


def _edge_b_lp(ev, eattr, tdst, tlig, W_eup, b_eup, W_flp, b_flp):
    epad = ev.shape[0]
    consts = [jnp.asarray(m) for m in (_SDEN_LP, _PL, _PB, _PNV, _R12, _C12)]
    return pl.pallas_call(
        _edge_b_lp_body,
        grid=(epad // _BLK_E,),
        in_specs=[_rows((_BLK_E, 128)), _rows((_BLK_E, 128)),
                  _rows((_BLK_E, 128)), _rows((_BLK_E, 128)),
                  _full((128, 128)), _full((128, 128)), _full((1, 128)),
                  _full((128, 128)), _full((4, 128)), _full((1, 128)),
                  _full((128, 128)), _full((128, 12)), _full((128, 12)),
                  _full((128, 12)), _full((12, 4)), _full((12, 1))],
        out_specs=_rows((_BLK_E, 128)),
        out_shape=jax.ShapeDtypeStruct((epad, 128), jnp.float32),
    )(ev, eattr, tdst, tlig, W_eup[:128], W_eup[128:], b_eup.reshape(1, 128),
      W_flp[:128], W_flp[128:], b_flp.reshape(1, 128), *consts)


def _pad_rows(x, n):
    return jnp.pad(x, ((0, n - x.shape[0]),) + ((0, 0),) * (x.ndim - 1))


@jax.jit
def _run(prot_scalars, prot_vectors, lig_scalars, lig_vectors, pr_pr_eattr,
         lig_pr_eattr, pr_pr_edge_index, lig_pr_edge_index, lig_coords,
         backbone_coords, W_msg0, b_msg0, a_attn0, W_val0, W_head0, W_eup0,
         b_eup0, W_msg1, b_msg1, a_attn1, W_val1, W_head1, W_eup1, b_eup1,
         W_n1, b_n1, W_n2, b_n2, W_gate, W_flp, b_flp, W_fpp, b_fpp):
    prot = _pad_rows(prot_scalars, N_PAD)
    tab1 = jnp.concatenate([prot, lig_scalars], axis=0)
    ep, el = pr_pr_edge_index, lig_pr_edge_index
    pps = jnp.pad(ep[0], (0, EPP_PAD - E_PP))
    ppd = jnp.pad(ep[1], (0, EPP_PAD - E_PP))
    lps = jnp.pad(el[0], (0, ELP_PAD - E_LP))
    lpd = jnp.pad(el[1], (0, ELP_PAD - E_LP))
    ppd_sc = jnp.pad(ep[1], (0, EPP_PAD - E_PP), constant_values=TRASH)
    lpd_sc = jnp.pad(el[1], (0, ELP_PAD - E_LP), constant_values=TRASH)

    idx1 = jnp.concatenate([pps, ppd, lps + N_PAD, lpd])
    g1 = _gather(tab1, idx1, 128)
    pp_src_g = g1[:EPP_PAD]
    pp_dst_g = g1[EPP_PAD:2 * EPP_PAD]
    lp_src_g = g1[2 * EPP_PAD:2 * EPP_PAD + ELP_PAD]
    lp_dst_g = g1[2 * EPP_PAD + ELP_PAD:]

    eattr_pp = _pad_rows(pr_pr_eattr, EPP_PAD)
    eattr_lp = _pad_rows(lig_pr_eattr, ELP_PAD)

    ev_pp, ex_pp = _edge_a(pp_src_g, pp_dst_g, eattr_pp, W_msg0, b_msg0, a_attn0, W_val0)
    ev_lp, ex_lp = _edge_a(lp_src_g, lp_dst_g, eattr_lp, W_msg1, b_msg1, a_attn1, W_val1)

    zeros = jnp.zeros((N_PAD, 128), jnp.float32)
    zeros_den = jnp.zeros((N_PAD * 4,), jnp.float32)
    idx4_pp = (ppd_sc[:, None] * 4 + jnp.arange(4, dtype=jnp.int32)[None, :])
    idx4_lp = (lpd_sc[:, None] * 4 + jnp.arange(4, dtype=jnp.int32)[None, :])
    npp, dpp, nlp, dlp = _scatter(
        ev_pp, ex_pp.reshape(-1, CHUNK),
        ppd_sc.reshape(-1, CHUNK), idx4_pp.reshape(-1, CHUNK),
        ev_lp, ex_lp.reshape(-1, CHUNK),
        lpd_sc.reshape(-1, CHUNK), idx4_lp.reshape(-1, CHUNK),
        zeros, zeros_den)
    dpp = dpp.reshape(2, N_PAD, 4)
    dlp = dlp.reshape(2, N_PAD, 4)

    pvec16 = jnp.pad(_pad_rows(prot_vectors.reshape(N_PR, 12), N_PAD),
                     ((0, 0), (0, 4)))
    bb16 = jnp.pad(_pad_rows(backbone_coords[:, 1], N_PAD), ((0, 0), (0, 13)))
    Wg16 = jnp.pad(W_gate, ((0, 0), (0, 12)))
    new_scal, nvec16, geom = _node(
        prot, pvec16, bb16, npp[:N_PAD], npp[N_PAD:], nlp[:N_PAD], nlp[N_PAD:],
        dpp[0], dpp[1], dlp[0], dlp[1],
        W_n1, b_n1, W_n2, b_n2, Wg16, W_head0, W_head1)

    ligtab = jnp.pad(lig_coords, ((0, 0), (0, 125)))
    tab2 = jnp.concatenate([geom, ligtab], axis=0)
    idx2 = jnp.concatenate([pps, ppd, lpd, lps + N_PAD])
    g2 = _gather(tab2, idx2, 128)
    tsrc_pp = g2[:EPP_PAD]
    tdst_pp = g2[EPP_PAD:2 * EPP_PAD]
    tdst_lp = g2[2 * EPP_PAD:2 * EPP_PAD + ELP_PAD]
    tlig = g2[2 * EPP_PAD + ELP_PAD:]

    pp_out = _edge_b_pp(ev_pp, eattr_pp, tsrc_pp, tdst_pp,
                        W_eup0, b_eup0, W_fpp, b_fpp)
    lp_out = _edge_b_lp(ev_lp, eattr_lp, tdst_lp, tlig,
                        W_eup1, b_eup1, W_flp, b_flp)

    return (new_scal[:N_PR],
            nvec16[:N_PR, :12].reshape(N_PR, NV, 3),
            pp_out[:E_PP],
            lp_out[:E_LP])


def kernel(*args):
    return _run(*args)


# speedup vs baseline: 2.3760x; 1.1372x over previous
import functools
import numpy as np
import jax
import jax.numpy as jnp
from jax import lax
from jax.experimental import pallas as pl
from jax.experimental.pallas import tpu as pltpu
from jax.experimental.pallas import tpu_sc as plsc

N_PR, N_LIG = 10000, 2000
E_PP, E_LP = 160000, 32000
NV, H = 4, 4

N_PAD = 10240
TRASH = 10000
EPP_PAD = 163840
ELP_PAD = 32768
NW = 32
CHUNK = 128


def _sel(shape, coords):
    m = np.zeros(shape, np.float32)
    for r, c in coords:
        m[r, c] = 1.0
    return m

_R_HEAD = _sel((128, 4), [(h * 32 + d, h) for h in range(4) for d in range(32)])
_P_HEAD = _sel((4, 128), [(h, h * 32 + d) for h in range(4) for d in range(32)])
_EYE4_16 = _sel((4, 16), [(i, i) for i in range(4)])
_SDEN_PP = _sel((128, 128), [(16 + h, h * 32 + d) for h in range(4) for d in range(32)])
_SDEN_LP = _sel((128, 128), [(20 + h, h * 32 + d) for h in range(4) for d in range(32)])
_P1X = _sel((128, 48), [(j * 3 + k, j * 12 + m * 3 + k) for j in range(4) for m in range(4) for k in range(3)])
_P2X = _sel((128, 48), [(m * 3 + k, j * 12 + m * 3 + k) for j in range(4) for m in range(4) for k in range(3)])
_R48 = _sel((48, 16), [(j * 12 + m * 3 + k, j * 4 + m) for j in range(4) for m in range(4) for k in range(3)])
_PL = _sel((128, 12), [(k, j * 3 + k) for j in range(4) for k in range(3)])
_PB = _sel((128, 12), [(24 + k, j * 3 + k) for j in range(4) for k in range(3)])
_PNV = _sel((128, 12), [(i, i) for i in range(12)])
_R12 = _sel((12, 4), [(j * 3 + k, j) for j in range(4) for k in range(3)])
_C12 = np.full((12, 1), 0.25, np.float32)
_G16 = _sel((16, 16), [(j, j * 3 + k) for j in range(4) for k in range(3)])
_N16 = _sel((16, 16), [(j * 3 + k, j) for j in range(4) for k in range(3)])
_K1 = _sel((16, 128), [(i, i) for i in range(12)])
_S4PP = _sel((4, 128), [(h, 16 + h) for h in range(4)])
_S4LP = _sel((4, 128), [(h, 20 + h) for h in range(4)])
_KB = _sel((16, 128), [(k, 24 + k) for k in range(3)])


def _gather(table, idx, width):
    etot = idx.shape[0]
    cpt = etot // (NW * CHUNK)
    mesh = plsc.VectorSubcoreMesh(core_axis_name="c", subcore_axis_name="s",
                                  num_cores=2)

    @functools.partial(
        pl.kernel, mesh=mesh,
        out_type=jax.ShapeDtypeStruct((etot, width), jnp.float32),
        scratch_types=[
            pltpu.VMEM((2, CHUNK), jnp.int32),
            pltpu.VMEM((2, CHUNK, width), jnp.float32),
            pltpu.SemaphoreType.DMA,
            pltpu.SemaphoreType.DMA,
            pltpu.SemaphoreType.DMA,
        ],
    )
    def k(tab_hbm, idx_hbm, out_hbm, idx_v, rows_v, sidx, sgat, sout):
        wid = lax.axis_index("c") * 16 + lax.axis_index("s")
        base = wid * (cpt * CHUNK)
        pltpu.sync_copy(idx_hbm.at[pl.ds(base, CHUNK)], idx_v.at[0])

        def body(j, _):
            slot = j % 2

            @pl.when(j >= 2)
            def _free_rows():
                pltpu.make_async_copy(
                    rows_v.at[slot], out_hbm.at[pl.ds(base, CHUNK)], sout
                ).wait()

            @pl.when(j >= 1)
            def _wait_idx():
                pltpu.make_async_copy(
                    idx_hbm.at[pl.ds(base, CHUNK)], idx_v.at[slot], sidx
                ).wait()

            pltpu.async_copy(tab_hbm.at[idx_v.at[slot]], rows_v.at[slot], sgat)

            @pl.when(j >= 1)
            def _retire_prev():
                pltpu.make_async_copy(
                    tab_hbm.at[idx_v.at[(j + 1) % 2]],
                    rows_v.at[(j + 1) % 2], sgat).wait()
                pltpu.async_copy(
                    rows_v.at[(j + 1) % 2],
                    out_hbm.at[pl.ds(base + (j - 1) * CHUNK, CHUNK)], sout)

            @pl.when(j + 1 < cpt)
            def _next_idx():
                pltpu.async_copy(
                    idx_hbm.at[pl.ds(base + (j + 1) * CHUNK, CHUNK)],
                    idx_v.at[(j + 1) % 2], sidx)

            return _

        lax.fori_loop(0, cpt, body, 0)
        last = (cpt - 1) % 2
        pltpu.make_async_copy(tab_hbm.at[idx_v.at[last]],
                              rows_v.at[last], sgat).wait()
        pltpu.async_copy(rows_v.at[last],
                         out_hbm.at[pl.ds(base + (cpt - 1) * CHUNK, CHUNK)],
                         sout)
        for _ in range(2):
            pltpu.make_async_copy(
                rows_v.at[0], out_hbm.at[pl.ds(base, CHUNK)], sout).wait()

    return k(table, idx)


def _scatter(ev_pp, ex_pp, idx_pp, idx4_pp, ev_lp, ex_lp, idx_lp, idx4_lp,
             zeros, zeros_den):
    cpt_pp = idx_pp.shape[0] // NW
    cpt_lp = idx_lp.shape[0] // NW
    zrows = N_PAD // 16
    dlen = (N_PAD * 4) // 16
    mesh = plsc.VectorSubcoreMesh(core_axis_name="c", subcore_axis_name="s",
                                  num_cores=2)

    @functools.partial(
        pl.kernel, mesh=mesh,
        out_type=[jax.ShapeDtypeStruct((2 * N_PAD, 128), jnp.float32),
                  jax.ShapeDtypeStruct((2, N_PAD * 4), jnp.float32),
                  jax.ShapeDtypeStruct((2 * N_PAD, 128), jnp.float32),
                  jax.ShapeDtypeStruct((2, N_PAD * 4), jnp.float32)],
        scratch_types=[
            pltpu.VMEM((8, CHUNK), jnp.int32),
            pltpu.VMEM((32, CHUNK), jnp.int32),
            pltpu.VMEM((32, CHUNK), jnp.float32),
            pltpu.VMEM((2, CHUNK, 128), jnp.float32),
            pltpu.SemaphoreType.DMA,
            pltpu.VMEM_SHARED((N_PAD, 128), jnp.float32),
            pltpu.VMEM_SHARED((N_PAD * 4,), jnp.float32),
        ],
    )
    def k(evp_hbm, exp_hbm, ip_hbm, i4p_hbm, evl_hbm, exl_hbm, il_hbm, i4l_hbm,
          z_hbm, zd_hbm, out_pp, outd_pp, out_lp, outd_lp,
          idx_v, idx4_v, ex_v, ev_v, sev, acc, accd):
        c = lax.axis_index("c")
        s = lax.axis_index("s")
        GR = 8

        def phase(ev_hbm, ex_hbm, idx_hbm, idx4_hbm, out_hbm, outd_hbm, cpt, half):
            pltpu.sync_copy(z_hbm.at[pl.ds(s * zrows, zrows)],
                            acc.at[pl.ds(s * zrows, zrows)])
            pltpu.sync_copy(zd_hbm.at[pl.ds(s * dlen, dlen)],
                            accd.at[pl.ds(s * dlen, dlen)])
            plsc.subcore_barrier()
            rowbase = c * half + s * cpt
            pltpu.sync_copy(ev_hbm.at[pl.ds(rowbase * CHUNK, CHUNK)],
                            ev_v.at[0])

            def group(g, _):
                base = rowbase + g * GR
                pltpu.sync_copy(idx_hbm.at[pl.ds(base, GR)], idx_v)
                pltpu.sync_copy(idx4_hbm.at[pl.ds(base * 4, GR * 4)], idx4_v)
                pltpu.sync_copy(ex_hbm.at[pl.ds(base * 4, GR * 4)], ex_v)
                for t in range(GR):
                    j = g * GR + t
                    slot = t % 2

                    @pl.when(j >= 1)
                    def _wait_ev():
                        pltpu.make_async_copy(
                            ev_hbm.at[pl.ds(rowbase * CHUNK, CHUNK)],
                            ev_v.at[slot], sev).wait()

                    @pl.when(j + 1 < cpt)
                    def _next_ev():
                        pltpu.async_copy(
                            ev_hbm.at[pl.ds((base + t + 1) * CHUNK, CHUNK)],
                            ev_v.at[(t + 1) % 2], sev)

                    pltpu.sync_copy(ev_v.at[slot], acc.at[idx_v.at[t]],
                                    add=True)
                    for r in range(4):
                        pltpu.sync_copy(ex_v.at[t * 4 + r],
                                        accd.at[idx4_v.at[t * 4 + r]],
                                        add=True)
                return _

            lax.fori_loop(0, cpt // GR, group, 0)
            plsc.subcore_barrier()
            pltpu.sync_copy(acc.at[pl.ds(s * zrows, zrows)],
                            out_hbm.at[pl.ds(c * N_PAD + s * zrows, zrows)])
            pltpu.sync_copy(accd.at[pl.ds(s * dlen, dlen)],
                            outd_hbm.at[c].at[pl.ds(s * dlen, dlen)])
            plsc.subcore_barrier()

        phase(evp_hbm, exp_hbm, ip_hbm, i4p_hbm, out_pp, outd_pp,
              cpt_pp, cpt_pp * 16)
        phase(evl_hbm, exl_hbm, il_hbm, i4l_hbm, out_lp, outd_lp,
              cpt_lp, cpt_lp * 16)

    return k(ev_pp, ex_pp, idx_pp, idx4_pp, ev_lp, ex_lp, idx_lp, idx4_lp,
             zeros, zeros_den)


_BLK_E = 2048
_BLK_N = 512


def _full(shape):
    return pl.BlockSpec(shape, lambda i: (0, 0))


def _rows(shape):
    return pl.BlockSpec(shape, lambda i: (i, 0))


def _edge_a_body(src_ref, dst_ref, ea_ref, w1, w2, w3, v1, v2, v3, bm, af,
                 rh, ph, out_ref, ex_ref):
    src, dst, ea = src_ref[...], dst_ref[...], ea_ref[...]
    pre = src @ w1[...] + dst @ w2[...] + ea @ w3[...] + bm[...]
    h = jnp.where(pre >= 0, pre, 0.2 * pre)
    ex4 = jnp.exp((h * af[...]) @ rh[...])
    vals = src @ v1[...] + dst @ v2[...] + ea @ v3[...]
    out_ref[...] = vals * (ex4 @ ph[...])
    ex_ref[...] = ex4


def _edge_a(srcg, dstg, eattr, W_msg, b_msg, a_attn, W_val):
    epad = srcg.shape[0]
    w = [W_msg[:128], W_msg[128:256], W_msg[256:384],
         W_val[:128], W_val[128:256], W_val[256:384]]
    consts = [b_msg.reshape(1, 128), a_attn.reshape(1, 128),
              jnp.asarray(_R_HEAD), jnp.asarray(_P_HEAD)]
    grid = epad // _BLK_E
    return pl.pallas_call(
        _edge_a_body,
        grid=(grid,),
        in_specs=[_rows((_BLK_E, 128))] * 3
        + [_full((128, 128))] * 6
        + [_full((1, 128)), _full((1, 128)), _full((128, 4)), _full((4, 128))],
        out_specs=[_rows((_BLK_E, 128)), _rows((_BLK_E, 4))],
        out_shape=[jax.ShapeDtypeStruct((epad, 128), jnp.float32),
                   jax.ShapeDtypeStruct((epad, 4), jnp.float32)],
    )(srcg, dstg, eattr, *w, *consts)


def _node_body(prot_ref, pvec_ref, bb_ref, npp0, npp1, nlp0, nlp1,
               dpp0, dpp1, dlp0, dlp1,
               a1, a2, a3, bn1, wn2, bn2, wg, wh0, wh1,
               ph, g16, n16, k1, s4pp, s4lp, kb,
               ns_ref, nvec_ref, geom_ref):
    prot = prot_ref[...]
    dpp = dpp0[...] + dpp1[...]
    dlp = dlp0[...] + dlp1[...]
    agg0 = (npp0[...] + npp1[...]) / (dpp @ ph[...] + 1e-9)
    agg1 = (nlp0[...] + nlp1[...]) / (dlp @ ph[...] + 1e-9)
    u = jnp.maximum(prot @ a1[...] + (agg0 @ wh0[...]) @ a2[...]
                    + (agg1 @ wh1[...]) @ a3[...] + bn1[...], 0.0)
    new_scal = prot + u @ wn2[...] + bn2[...]
    g = jax.nn.sigmoid(new_scal @ wg[...])
    nvec = pvec_ref[...] * (g @ g16[...])
    n2 = (nvec * nvec) @ n16[...]
    rn = 1.0 / jnp.sqrt(jnp.maximum(n2, 1e-8))
    nv = nvec * (rn @ g16[...])
    ns_ref[...] = new_scal
    nvec_ref[...] = nvec
    geom_ref[...] = (nv @ k1[...] + dpp @ s4pp[...] + dlp @ s4lp[...]
                     + bb_ref[...] @ kb[...])


def _node(prot, pvec16, bb16, npp0, npp1, nlp0, nlp1, dpp0, dpp1, dlp0, dlp1,
          W_n1, b_n1, W_n2, b_n2, Wg16, Wh0, Wh1):
    grid = N_PAD // _BLK_N
    consts = [jnp.asarray(m) for m in
              (_P_HEAD, _G16, _N16, _K1, _S4PP, _S4LP, _KB)]
    return pl.pallas_call(
        _node_body,
        grid=(grid,),
        in_specs=[_rows((_BLK_N, 128)), _rows((_BLK_N, 16)), _rows((_BLK_N, 16))]
        + [_rows((_BLK_N, 128))] * 4
        + [_rows((_BLK_N, 4))] * 4
        + [_full((128, 128))] * 3
        + [_full((1, 128)), _full((128, 128)), _full((1, 128)),
           _full((128, 16)), _full((128, 128)), _full((128, 128))]
        + [_full((4, 128)), _full((16, 16)), _full((16, 16)),
           _full((16, 128)), _full((4, 128)), _full((4, 128)), _full((16, 128))],
        out_specs=[_rows((_BLK_N, 128)), _rows((_BLK_N, 16)), _rows((_BLK_N, 128))],
        out_shape=[jax.ShapeDtypeStruct((N_PAD, 128), jnp.float32),
                   jax.ShapeDtypeStruct((N_PAD, 16), jnp.float32),
                   jax.ShapeDtypeStruct((N_PAD, 128), jnp.float32)],
    )(prot, pvec16, bb16, npp0, npp1, nlp0, nlp1, dpp0, dpp1, dlp0, dlp1,
      W_n1[:128], W_n1[128:256], W_n1[256:384],
      b_n1.reshape(1, 128), W_n2, b_n2.reshape(1, 128),
      Wg16, Wh0, Wh1, *consts)


def _edge_b_pp_body(ev_ref, ea_ref, ts_ref, td_ref, e1, e2, be, wf1, wf2, bf,
                    sden, p1x, p2x, r48, out_ref):
    ev, ea, ts, td = ev_ref[...], ea_ref[...], ts_ref[...], td_ref[...]
    w = ev / (td @ sden[...] + 1e-9)
    new_pp = jnp.maximum(ea @ e1[...] + w @ e2[...] + be[...], 0.0)
    t48 = (ts @ p1x[...]) * (td @ p2x[...])
    out_ref[...] = new_pp @ wf1[...] + (t48 @ r48[...]) @ wf2[...] + bf[...]


def _edge_b_pp(ev, eattr, tsrc, tdst, W_eup, b_eup, W_fpp, b_fpp):
    epad = ev.shape[0]
    consts = [jnp.asarray(m) for m in (_SDEN_PP, _P1X, _P2X, _R48)]
    return pl.pallas_call(
        _edge_b_pp_body,
        grid=(epad // _BLK_E,),
        in_specs=[_rows((_BLK_E, 128)), _rows((_BLK_E, 128)),
                  _rows((_BLK_E, 128)), _rows((_BLK_E, 128)),
                  _full((128, 128)), _full((128, 128)), _full((1, 128)),
                  _full((128, 128)), _full((16, 128)), _full((1, 128)),
                  _full((128, 128)), _full((128, 48)), _full((128, 48)),
                  _full((48, 16))],
        out_specs=_rows((_BLK_E, 128)),
        out_shape=jax.ShapeDtypeStruct((epad, 128), jnp.float32),
    )(ev, eattr, tsrc, tdst, W_eup[:128], W_eup[128:], b_eup.reshape(1, 128),
      W_fpp[:128], W_fpp[128:], b_fpp.reshape(1, 128), *consts)


def _edge_b_lp_body(ev_ref, ea_ref, td_ref, tl_ref, e1, e2, be, wf1, wf2, bf,
                    sden, plm, pbm, pnv, r12, c12, out_ref):
    ev, ea, td, tl = ev_ref[...], ea_ref[...], td_ref[...], tl_ref[...]
    w = ev / (td @ sden[...] + 1e-9)
    new_lp = jnp.maximum(ea @ e1[...] + w @ e2[...] + be[...], 0.0)
    d12 = tl @ plm[...] - td @ pbm[...]
    n2 = (d12 * d12) @ c12[...]
    rn = 1.0 / jnp.sqrt(jnp.maximum(n2, 1e-8))
    t12 = (td @ pnv[...]) * (d12 * rn)
    out_ref[...] = new_lp @ wf1[...] + (t12 @ r12[...]) @ wf2[...] + bf[...]


def _edge_b_lp(ev, eattr, tdst, tlig, W_eup, b_eup, W_flp, b_flp):
    epad = ev.shape[0]
    consts = [jnp.asarray(m) for m in (_SDEN_LP, _PL, _PB, _PNV, _R12, _C12)]
    return pl.pallas_call(
        _edge_b_lp_body,
        grid=(epad // _BLK_E,),
        in_specs=[_rows((_BLK_E, 128)), _rows((_BLK_E, 128)),
                  _rows((_BLK_E, 128)), _rows((_BLK_E, 128)),
                  _full((128, 128)), _full((128, 128)), _full((1, 128)),
                  _full((128, 128)), _full((4, 128)), _full((1, 128)),
                  _full((128, 128)), _full((128, 12)), _full((128, 12)),
                  _full((128, 12)), _full((12, 4)), _full((12, 1))],
        out_specs=_rows((_BLK_E, 128)),
        out_shape=jax.ShapeDtypeStruct((epad, 128), jnp.float32),
    )(ev, eattr, tdst, tlig, W_eup[:128], W_eup[128:], b_eup.reshape(1, 128),
      W_flp[:128], W_flp[128:], b_flp.reshape(1, 128), *consts)


def _pad_rows(x, n):
    return jnp.pad(x, ((0, n - x.shape[0]),) + ((0, 0),) * (x.ndim - 1))


@jax.jit
def _run(prot_scalars, prot_vectors, lig_scalars, lig_vectors, pr_pr_eattr,
         lig_pr_eattr, pr_pr_edge_index, lig_pr_edge_index, lig_coords,
         backbone_coords, W_msg0, b_msg0, a_attn0, W_val0, W_head0, W_eup0,
         b_eup0, W_msg1, b_msg1, a_attn1, W_val1, W_head1, W_eup1, b_eup1,
         W_n1, b_n1, W_n2, b_n2, W_gate, W_flp, b_flp, W_fpp, b_fpp):
    prot = _pad_rows(prot_scalars, N_PAD)
    tab1 = jnp.concatenate([prot, lig_scalars], axis=0)
    ep, el = pr_pr_edge_index, lig_pr_edge_index
    pps = jnp.pad(ep[0], (0, EPP_PAD - E_PP))
    ppd = jnp.pad(ep[1], (0, EPP_PAD - E_PP))
    lps = jnp.pad(el[0], (0, ELP_PAD - E_LP))
    lpd = jnp.pad(el[1], (0, ELP_PAD - E_LP))
    ppd_sc = jnp.pad(ep[1], (0, EPP_PAD - E_PP), constant_values=TRASH)
    lpd_sc = jnp.pad(el[1], (0, ELP_PAD - E_LP), constant_values=TRASH)

    g1pp = _gather(tab1, jnp.concatenate([pps, ppd]), 128)
    g1lp = _gather(tab1, jnp.concatenate([lps + N_PAD, lpd]), 128)
    pp_src_g = g1pp[:EPP_PAD]
    pp_dst_g = g1pp[EPP_PAD:]
    lp_src_g = g1lp[:ELP_PAD]
    lp_dst_g = g1lp[ELP_PAD:]

    eattr_pp = _pad_rows(pr_pr_eattr, EPP_PAD)
    eattr_lp = _pad_rows(lig_pr_eattr, ELP_PAD)

    ev_pp, ex_pp = _edge_a(pp_src_g, pp_dst_g, eattr_pp, W_msg0, b_msg0, a_attn0, W_val0)
    ev_lp, ex_lp = _edge_a(lp_src_g, lp_dst_g, eattr_lp, W_msg1, b_msg1, a_attn1, W_val1)

    zeros = jnp.zeros((N_PAD, 128), jnp.float32)
    zeros_den = jnp.zeros((N_PAD * 4,), jnp.float32)
    idx4_pp = (ppd_sc[:, None] * 4 + jnp.arange(4, dtype=jnp.int32)[None, :])
    idx4_lp = (lpd_sc[:, None] * 4 + jnp.arange(4, dtype=jnp.int32)[None, :])
    npp, dpp, nlp, dlp = _scatter(
        ev_pp, ex_pp.reshape(-1, CHUNK),
        ppd_sc.reshape(-1, CHUNK), idx4_pp.reshape(-1, CHUNK),
        ev_lp, ex_lp.reshape(-1, CHUNK),
        lpd_sc.reshape(-1, CHUNK), idx4_lp.reshape(-1, CHUNK),
        zeros, zeros_den)
    dpp = dpp.reshape(2, N_PAD, 4)
    dlp = dlp.reshape(2, N_PAD, 4)

    pvec16 = jnp.pad(_pad_rows(prot_vectors.reshape(N_PR, 12), N_PAD),
                     ((0, 0), (0, 4)))
    bb16 = jnp.pad(_pad_rows(backbone_coords[:, 1], N_PAD), ((0, 0), (0, 13)))
    Wg16 = jnp.pad(W_gate, ((0, 0), (0, 12)))
    new_scal, nvec16, geom = _node(
        prot, pvec16, bb16, npp[:N_PAD], npp[N_PAD:], nlp[:N_PAD], nlp[N_PAD:],
        dpp[0], dpp[1], dlp[0], dlp[1],
        W_n1, b_n1, W_n2, b_n2, Wg16, W_head0, W_head1)

    ligtab = jnp.pad(lig_coords, ((0, 0), (0, 125)))
    tab2 = jnp.concatenate([geom, ligtab], axis=0)
    g2pp = _gather(tab2, jnp.concatenate([pps, ppd]), 128)
    g2lp = _gather(tab2, jnp.concatenate([lpd, lps + N_PAD]), 128)
    tsrc_pp = g2pp[:EPP_PAD]
    tdst_pp = g2pp[EPP_PAD:]
    tdst_lp = g2lp[:ELP_PAD]
    tlig = g2lp[ELP_PAD:]

    pp_out = _edge_b_pp(ev_pp, eattr_pp, tsrc_pp, tdst_pp,
                        W_eup0, b_eup0, W_fpp, b_fpp)
    lp_out = _edge_b_lp(ev_lp, eattr_lp, tdst_lp, tlig,
                        W_eup1, b_eup1, W_flp, b_flp)

    return (new_scal[:N_PR],
            nvec16[:N_PR, :12].reshape(N_PR, NV, 3),
            pp_out[:E_PP],
            lp_out[:E_LP])


def kernel(*args):
    return _run(*args)


# speedup vs baseline: 2.7905x; 1.1744x over previous
import functools
import numpy as np
import jax
import jax.numpy as jnp
from jax import lax
from jax.experimental import pallas as pl
from jax.experimental.pallas import tpu as pltpu
from jax.experimental.pallas import tpu_sc as plsc

N_PR, N_LIG = 10000, 2000
E_PP, E_LP = 160000, 32000
NV, H = 4, 4

N_PAD = 10240
TRASH = 10000
EPP_PAD = 163840
ELP_PAD = 32768
NW = 32
CHUNK = 128


def _sel(shape, coords):
    m = np.zeros(shape, np.float32)
    for r, c in coords:
        m[r, c] = 1.0
    return m

_R_HEAD = _sel((128, 4), [(h * 32 + d, h) for h in range(4) for d in range(32)])
_P_HEAD = _sel((4, 128), [(h, h * 32 + d) for h in range(4) for d in range(32)])
_EYE4_16 = _sel((4, 16), [(i, i) for i in range(4)])
_SDEN_PP = _sel((128, 128), [(16 + h, h * 32 + d) for h in range(4) for d in range(32)])
_SDEN_LP = _sel((128, 128), [(20 + h, h * 32 + d) for h in range(4) for d in range(32)])
_P1X = _sel((128, 48), [(j * 3 + k, j * 12 + m * 3 + k) for j in range(4) for m in range(4) for k in range(3)])
_P2X = _sel((128, 48), [(m * 3 + k, j * 12 + m * 3 + k) for j in range(4) for m in range(4) for k in range(3)])
_R48 = _sel((48, 16), [(j * 12 + m * 3 + k, j * 4 + m) for j in range(4) for m in range(4) for k in range(3)])
_PL = _sel((128, 12), [(k, j * 3 + k) for j in range(4) for k in range(3)])
_PB = _sel((128, 12), [(24 + k, j * 3 + k) for j in range(4) for k in range(3)])
_PNV = _sel((128, 12), [(i, i) for i in range(12)])
_R12 = _sel((12, 4), [(j * 3 + k, j) for j in range(4) for k in range(3)])
_C12 = np.full((12, 1), 0.25, np.float32)
_G16 = _sel((16, 16), [(j, j * 3 + k) for j in range(4) for k in range(3)])
_N16 = _sel((16, 16), [(j * 3 + k, j) for j in range(4) for k in range(3)])
_K1 = _sel((16, 128), [(i, i) for i in range(12)])
_S4PP = _sel((4, 128), [(h, 16 + h) for h in range(4)])
_S4LP = _sel((4, 128), [(h, 20 + h) for h in range(4)])
_KB = _sel((16, 128), [(k, 24 + k) for k in range(3)])


def _gather(table, idx, width):
    etot = idx.shape[0]
    cpt = etot // (NW * CHUNK)
    mesh = plsc.VectorSubcoreMesh(core_axis_name="c", subcore_axis_name="s",
                                  num_cores=2)

    @functools.partial(
        pl.kernel, mesh=mesh,
        out_type=jax.ShapeDtypeStruct((etot, width), jnp.float32),
        scratch_types=[
            pltpu.VMEM((2, CHUNK), jnp.int32),
            pltpu.VMEM((2, CHUNK, width), jnp.float32),
            pltpu.SemaphoreType.DMA,
            pltpu.SemaphoreType.DMA,
            pltpu.SemaphoreType.DMA,
        ],
    )
    def k(tab_hbm, idx_hbm, out_hbm, idx_v, rows_v, sidx, sgat, sout):
        wid = lax.axis_index("c") * 16 + lax.axis_index("s")
        base = wid * (cpt * CHUNK)
        pltpu.sync_copy(idx_hbm.at[pl.ds(base, CHUNK)], idx_v.at[0])

        def body(j, _):
            slot = j % 2

            @pl.when(j >= 2)
            def _free_rows():
                pltpu.make_async_copy(
                    rows_v.at[slot], out_hbm.at[pl.ds(base, CHUNK)], sout
                ).wait()

            @pl.when(j >= 1)
            def _wait_idx():
                pltpu.make_async_copy(
                    idx_hbm.at[pl.ds(base, CHUNK)], idx_v.at[slot], sidx
                ).wait()

            pltpu.async_copy(tab_hbm.at[idx_v.at[slot]], rows_v.at[slot], sgat)

            @pl.when(j >= 1)
            def _retire_prev():
                pltpu.make_async_copy(
                    tab_hbm.at[idx_v.at[(j + 1) % 2]],
                    rows_v.at[(j + 1) % 2], sgat).wait()
                pltpu.async_copy(
                    rows_v.at[(j + 1) % 2],
                    out_hbm.at[pl.ds(base + (j - 1) * CHUNK, CHUNK)], sout)

            @pl.when(j + 1 < cpt)
            def _next_idx():
                pltpu.async_copy(
                    idx_hbm.at[pl.ds(base + (j + 1) * CHUNK, CHUNK)],
                    idx_v.at[(j + 1) % 2], sidx)

            return _

        lax.fori_loop(0, cpt, body, 0)
        last = (cpt - 1) % 2
        pltpu.make_async_copy(tab_hbm.at[idx_v.at[last]],
                              rows_v.at[last], sgat).wait()
        pltpu.async_copy(rows_v.at[last],
                         out_hbm.at[pl.ds(base + (cpt - 1) * CHUNK, CHUNK)],
                         sout)
        for _ in range(2):
            pltpu.make_async_copy(
                rows_v.at[0], out_hbm.at[pl.ds(base, CHUNK)], sout).wait()

    return k(table, idx)


def _scatter(ev_pp, ex_pp, idx_pp, idx4_pp, ev_lp, ex_lp, idx_lp, idx4_lp,
             zeros, zeros_den):
    cpt_pp = idx_pp.shape[0] // NW
    cpt_lp = idx_lp.shape[0] // NW
    zrows = N_PAD // 16
    dlen = (N_PAD * 4) // 16
    mesh = plsc.VectorSubcoreMesh(core_axis_name="c", subcore_axis_name="s",
                                  num_cores=2)

    @functools.partial(
        pl.kernel, mesh=mesh,
        out_type=[jax.ShapeDtypeStruct((2 * N_PAD, 128), jnp.float32),
                  jax.ShapeDtypeStruct((2, N_PAD * 4), jnp.float32),
                  jax.ShapeDtypeStruct((2 * N_PAD, 128), jnp.float32),
                  jax.ShapeDtypeStruct((2, N_PAD * 4), jnp.float32)],
        scratch_types=[
            pltpu.VMEM((8, CHUNK), jnp.int32),
            pltpu.VMEM((32, CHUNK), jnp.int32),
            pltpu.VMEM((32, CHUNK), jnp.float32),
            pltpu.VMEM((2, CHUNK, 128), jnp.float32),
            pltpu.SemaphoreType.DMA,
            pltpu.VMEM_SHARED((N_PAD, 128), jnp.float32),
            pltpu.VMEM_SHARED((N_PAD * 4,), jnp.float32),
        ],
    )
    def k(evp_hbm, exp_hbm, ip_hbm, i4p_hbm, evl_hbm, exl_hbm, il_hbm, i4l_hbm,
          z_hbm, zd_hbm, out_pp, outd_pp, out_lp, outd_lp,
          idx_v, idx4_v, ex_v, ev_v, sev, acc, accd):
        c = lax.axis_index("c")
        s = lax.axis_index("s")
        GR = 8

        def phase(ev_hbm, ex_hbm, idx_hbm, idx4_hbm, out_hbm, outd_hbm, cpt, half):
            pltpu.sync_copy(z_hbm.at[pl.ds(s * zrows, zrows)],
                            acc.at[pl.ds(s * zrows, zrows)])
            pltpu.sync_copy(zd_hbm.at[pl.ds(s * dlen, dlen)],
                            accd.at[pl.ds(s * dlen, dlen)])
            plsc.subcore_barrier()
            rowbase = c * half + s * cpt
            pltpu.sync_copy(ev_hbm.at[pl.ds(rowbase * CHUNK, CHUNK)],
                            ev_v.at[0])

            def group(g, _):
                base = rowbase + g * GR
                pltpu.sync_copy(idx_hbm.at[pl.ds(base, GR)], idx_v)
                pltpu.sync_copy(idx4_hbm.at[pl.ds(base * 4, GR * 4)], idx4_v)
                pltpu.sync_copy(ex_hbm.at[pl.ds(base * 4, GR * 4)], ex_v)
                for t in range(GR):
                    j = g * GR + t
                    slot = t % 2

                    @pl.when(j >= 1)
                    def _wait_ev():
                        pltpu.make_async_copy(
                            ev_hbm.at[pl.ds(rowbase * CHUNK, CHUNK)],
                            ev_v.at[slot], sev).wait()

                    @pl.when(j + 1 < cpt)
                    def _next_ev():
                        pltpu.async_copy(
                            ev_hbm.at[pl.ds((base + t + 1) * CHUNK, CHUNK)],
                            ev_v.at[(t + 1) % 2], sev)

                    pltpu.sync_copy(ev_v.at[slot], acc.at[idx_v.at[t]],
                                    add=True)
                    for r in range(4):
                        pltpu.sync_copy(ex_v.at[t * 4 + r],
                                        accd.at[idx4_v.at[t * 4 + r]],
                                        add=True)
                return _

            lax.fori_loop(0, cpt // GR, group, 0)
            plsc.subcore_barrier()
            pltpu.sync_copy(acc.at[pl.ds(s * zrows, zrows)],
                            out_hbm.at[pl.ds(c * N_PAD + s * zrows, zrows)])
            pltpu.sync_copy(accd.at[pl.ds(s * dlen, dlen)],
                            outd_hbm.at[c].at[pl.ds(s * dlen, dlen)])
            plsc.subcore_barrier()

        phase(evp_hbm, exp_hbm, ip_hbm, i4p_hbm, out_pp, outd_pp,
              cpt_pp, cpt_pp * 16)
        phase(evl_hbm, exl_hbm, il_hbm, i4l_hbm, out_lp, outd_lp,
              cpt_lp, cpt_lp * 16)

    return k(ev_pp, ex_pp, idx_pp, idx4_pp, ev_lp, ex_lp, idx_lp, idx4_lp,
             zeros, zeros_den)


_BLK_E = 2048
_BLK_N = 512


def _full(shape):
    return pl.BlockSpec(shape, lambda i: (0, 0))


def _rows(shape):
    return pl.BlockSpec(shape, lambda i: (i, 0))


def _edge_a_body(src_ref, dst_ref, ea_ref, w1, w2, w3, v1, v2, v3, bm, af,
                 rh, ph, out_ref, ex_ref):
    src, dst, ea = src_ref[...], dst_ref[...], ea_ref[...]
    pre = src @ w1[...] + dst @ w2[...] + ea @ w3[...] + bm[...]
    h = jnp.where(pre >= 0, pre, 0.2 * pre)
    ex4 = jnp.exp((h * af[...]) @ rh[...])
    vals = src @ v1[...] + dst @ v2[...] + ea @ v3[...]
    out_ref[...] = vals * (ex4 @ ph[...])
    ex_ref[...] = ex4


def _edge_a(g1, eattr, W_msg, b_msg, a_attn, W_val):
    epad = eattr.shape[0]
    w = [W_msg[:128], W_msg[128:256], W_msg[256:384],
         W_val[:128], W_val[128:256], W_val[256:384]]
    consts = [b_msg.reshape(1, 128), a_attn.reshape(1, 128),
              jnp.asarray(_R_HEAD), jnp.asarray(_P_HEAD)]
    grid = epad // _BLK_E
    off = epad // _BLK_E
    return pl.pallas_call(
        _edge_a_body,
        grid=(grid,),
        in_specs=[_rows((_BLK_E, 128)),
                  pl.BlockSpec((_BLK_E, 128), lambda i: (i + off, 0)),
                  _rows((_BLK_E, 128))]
        + [_full((128, 128))] * 6
        + [_full((1, 128)), _full((1, 128)), _full((128, 4)), _full((4, 128))],
        out_specs=[_rows((_BLK_E, 128)), _rows((_BLK_E, 4))],
        out_shape=[jax.ShapeDtypeStruct((epad, 128), jnp.float32),
                   jax.ShapeDtypeStruct((epad, 4), jnp.float32)],
    )(g1, g1, eattr, *w, *consts)


def _node_body(prot_ref, pvec_ref, bb_ref, npp0, npp1, nlp0, nlp1,
               dpp0, dpp1, dlp0, dlp1,
               a1, a2, a3, bn1, wn2, bn2, wg, wh0, wh1,
               ph, g16, n16, k1, s4pp, s4lp, kb,
               ns_ref, nvec_ref, geom_ref):
    prot = prot_ref[...]
    dpp = dpp0[...] + dpp1[...]
    dlp = dlp0[...] + dlp1[...]
    agg0 = (npp0[...] + npp1[...]) / (dpp @ ph[...] + 1e-9)
    agg1 = (nlp0[...] + nlp1[...]) / (dlp @ ph[...] + 1e-9)
    u = jnp.maximum(prot @ a1[...] + (agg0 @ wh0[...]) @ a2[...]
                    + (agg1 @ wh1[...]) @ a3[...] + bn1[...], 0.0)
    new_scal = prot + u @ wn2[...] + bn2[...]
    g = jax.nn.sigmoid(new_scal @ wg[...])
    nvec = pvec_ref[...] * (g @ g16[...])
    n2 = (nvec * nvec) @ n16[...]
    rn = 1.0 / jnp.sqrt(jnp.maximum(n2, 1e-8))
    nv = nvec * (rn @ g16[...])
    ns_ref[...] = new_scal
    nvec_ref[...] = nvec
    geom_ref[...] = (nv @ k1[...] + dpp @ s4pp[...] + dlp @ s4lp[...]
                     + bb_ref[...] @ kb[...])


def _node(prot, pvec16, bb16, npp0, npp1, nlp0, nlp1, dpp0, dpp1, dlp0, dlp1,
          W_n1, b_n1, W_n2, b_n2, Wg16, Wh0, Wh1):
    grid = N_PAD // _BLK_N
    consts = [jnp.asarray(m) for m in
              (_P_HEAD, _G16, _N16, _K1, _S4PP, _S4LP, _KB)]
    return pl.pallas_call(
        _node_body,
        grid=(grid,),
        in_specs=[_rows((_BLK_N, 128)), _rows((_BLK_N, 16)), _rows((_BLK_N, 16))]
        + [_rows((_BLK_N, 128))] * 4
        + [_rows((_BLK_N, 4))] * 4
        + [_full((128, 128))] * 3
        + [_full((1, 128)), _full((128, 128)), _full((1, 128)),
           _full((128, 16)), _full((128, 128)), _full((128, 128))]
        + [_full((4, 128)), _full((16, 16)), _full((16, 16)),
           _full((16, 128)), _full((4, 128)), _full((4, 128)), _full((16, 128))],
        out_specs=[_rows((_BLK_N, 128)), _rows((_BLK_N, 16)), _rows((_BLK_N, 128))],
        out_shape=[jax.ShapeDtypeStruct((N_PR, 128), jnp.float32),
                   jax.ShapeDtypeStruct((N_PR, 16), jnp.float32),
                   jax.ShapeDtypeStruct((N_PAD, 128), jnp.float32)],
    )(prot, pvec16, bb16, npp0, npp1, nlp0, nlp1, dpp0, dpp1, dlp0, dlp1,
      W_n1[:128], W_n1[128:256], W_n1[256:384],
      b_n1.reshape(1, 128), W_n2, b_n2.reshape(1, 128),
      Wg16, Wh0, Wh1, *consts)


def _edge_b_pp_body(ev_ref, ea_ref, ts_ref, td_ref, e1, e2, be, wf1, wf2, bf,
                    sden, p1x, p2x, r48, out_ref):
    ev, ea, ts, td = ev_ref[...], ea_ref[...], ts_ref[...], td_ref[...]
    w = ev / (td @ sden[...] + 1e-9)
    new_pp = jnp.maximum(ea @ e1[...] + w @ e2[...] + be[...], 0.0)
    t48 = (ts @ p1x[...]) * (td @ p2x[...])
    out_ref[...] = new_pp @ wf1[...] + (t48 @ r48[...]) @ wf2[...] + bf[...]


def _edge_b_pp(ev, eattr, g2, n_out, W_eup, b_eup, W_fpp, b_fpp):
    epad = ev.shape[0]
    off = epad // _BLK_E
    consts = [jnp.asarray(m) for m in (_SDEN_PP, _P1X, _P2X, _R48)]
    return pl.pallas_call(
        _edge_b_pp_body,
        grid=(pl.cdiv(n_out, _BLK_E),),
        in_specs=[_rows((_BLK_E, 128)), _rows((_BLK_E, 128)),
                  _rows((_BLK_E, 128)),
                  pl.BlockSpec((_BLK_E, 128), lambda i: (i + off, 0)),
                  _full((128, 128)), _full((128, 128)), _full((1, 128)),
                  _full((128, 128)), _full((16, 128)), _full((1, 128)),
                  _full((128, 128)), _full((128, 48)), _full((128, 48)),
                  _full((48, 16))],
        out_specs=_rows((_BLK_E, 128)),
        out_shape=jax.ShapeDtypeStruct((n_out, 128), jnp.float32),
    )(ev, eattr, g2, g2, W_eup[:128], W_eup[128:], b_eup.reshape(1, 128),
      W_fpp[:128], W_fpp[128:], b_fpp.reshape(1, 128), *consts)


def _edge_b_lp_body(ev_ref, ea_ref, td_ref, tl_ref, e1, e2, be, wf1, wf2, bf,
                    sden, plm, pbm, pnv, r12, c12, out_ref):
    ev, ea, td, tl = ev_ref[...], ea_ref[...], td_ref[...], tl_ref[...]
    w = ev / (td @ sden[...] + 1e-9)
    new_lp = jnp.maximum(ea @ e1[...] + w @ e2[...] + be[...], 0.0)
    d12 = tl @ plm[...] - td @ pbm[...]
    n2 = (d12 * d12) @ c12[...]
    rn = 1.0 / jnp.sqrt(jnp.maximum(n2, 1e-8))
    t12 = (td @ pnv[...]) * (d12 * rn)
    out_ref[...] = new_lp @ wf1[...] + (t12 @ r12[...]) @ wf2[...] + bf[...]


def _edge_b_lp(ev, eattr, g2, n_out, W_eup, b_eup, W_flp, b_flp):
    epad = ev.shape[0]
    off = epad // _BLK_E
    consts = [jnp.asarray(m) for m in (_SDEN_LP, _PL, _PB, _PNV, _R12, _C12)]
    return pl.pallas_call(
        _edge_b_lp_body,
        grid=(pl.cdiv(n_out, _BLK_E),),
        in_specs=[_rows((_BLK_E, 128)), _rows((_BLK_E, 128)),
                  _rows((_BLK_E, 128)),
                  pl.BlockSpec((_BLK_E, 128), lambda i: (i + off, 0)),
                  _full((128, 128)), _full((128, 128)), _full((1, 128)),
                  _full((128, 128)), _full((4, 128)), _full((1, 128)),
                  _full((128, 128)), _full((128, 12)), _full((128, 12)),
                  _full((128, 12)), _full((12, 4)), _full((12, 1))],
        out_specs=_rows((_BLK_E, 128)),
        out_shape=jax.ShapeDtypeStruct((n_out, 128), jnp.float32),
    )(ev, eattr, g2, g2, W_eup[:128], W_eup[128:], b_eup.reshape(1, 128),
      W_flp[:128], W_flp[128:], b_flp.reshape(1, 128), *consts)


def _pad_rows(x, n):
    return jnp.pad(x, ((0, n - x.shape[0]),) + ((0, 0),) * (x.ndim - 1))


@jax.jit
def _run(prot_scalars, prot_vectors, lig_scalars, lig_vectors, pr_pr_eattr,
         lig_pr_eattr, pr_pr_edge_index, lig_pr_edge_index, lig_coords,
         backbone_coords, W_msg0, b_msg0, a_attn0, W_val0, W_head0, W_eup0,
         b_eup0, W_msg1, b_msg1, a_attn1, W_val1, W_head1, W_eup1, b_eup1,
         W_n1, b_n1, W_n2, b_n2, W_gate, W_flp, b_flp, W_fpp, b_fpp):
    prot = _pad_rows(prot_scalars, N_PAD)
    tab1 = jnp.concatenate([prot, lig_scalars], axis=0)
    ep, el = pr_pr_edge_index, lig_pr_edge_index
    pps = jnp.pad(ep[0], (0, EPP_PAD - E_PP))
    ppd = jnp.pad(ep[1], (0, EPP_PAD - E_PP))
    lps = jnp.pad(el[0], (0, ELP_PAD - E_LP))
    lpd = jnp.pad(el[1], (0, ELP_PAD - E_LP))
    ppd_sc = jnp.pad(ep[1], (0, EPP_PAD - E_PP), constant_values=TRASH)
    lpd_sc = jnp.pad(el[1], (0, ELP_PAD - E_LP), constant_values=TRASH)

    g1pp = _gather(tab1, jnp.concatenate([pps, ppd]), 128)
    g1lp = _gather(tab1, jnp.concatenate([lps + N_PAD, lpd]), 128)

    eattr_pp = _pad_rows(pr_pr_eattr, EPP_PAD)
    eattr_lp = _pad_rows(lig_pr_eattr, ELP_PAD)

    ev_pp, ex_pp = _edge_a(g1pp, eattr_pp, W_msg0, b_msg0, a_attn0, W_val0)
    ev_lp, ex_lp = _edge_a(g1lp, eattr_lp, W_msg1, b_msg1, a_attn1, W_val1)

    zeros = jnp.zeros((N_PAD, 128), jnp.float32)
    zeros_den = jnp.zeros((N_PAD * 4,), jnp.float32)
    idx4_pp = (ppd_sc[:, None] * 4 + jnp.arange(4, dtype=jnp.int32)[None, :])
    idx4_lp = (lpd_sc[:, None] * 4 + jnp.arange(4, dtype=jnp.int32)[None, :])
    npp, dpp, nlp, dlp = _scatter(
        ev_pp, ex_pp.reshape(-1, CHUNK),
        ppd_sc.reshape(-1, CHUNK), idx4_pp.reshape(-1, CHUNK),
        ev_lp, ex_lp.reshape(-1, CHUNK),
        lpd_sc.reshape(-1, CHUNK), idx4_lp.reshape(-1, CHUNK),
        zeros, zeros_den)
    dpp = dpp.reshape(2, N_PAD, 4)
    dlp = dlp.reshape(2, N_PAD, 4)

    pvec16 = jnp.pad(_pad_rows(prot_vectors.reshape(N_PR, 12), N_PAD),
                     ((0, 0), (0, 4)))
    bb16 = jnp.pad(_pad_rows(backbone_coords[:, 1], N_PAD), ((0, 0), (0, 13)))
    Wg16 = jnp.pad(W_gate, ((0, 0), (0, 12)))
    new_scal, nvec16, geom = _node(
        prot, pvec16, bb16, npp[:N_PAD], npp[N_PAD:], nlp[:N_PAD], nlp[N_PAD:],
        dpp[0], dpp[1], dlp[0], dlp[1],
        W_n1, b_n1, W_n2, b_n2, Wg16, W_head0, W_head1)

    ligtab = jnp.pad(lig_coords, ((0, 0), (0, 125)))
    tab2 = jnp.concatenate([geom, ligtab], axis=0)
    g2pp = _gather(tab2, jnp.concatenate([pps, ppd]), 128)
    g2lp = _gather(tab2, jnp.concatenate([lpd, lps + N_PAD]), 128)

    pp_out = _edge_b_pp(ev_pp, eattr_pp, g2pp, E_PP,
                        W_eup0, b_eup0, W_fpp, b_fpp)
    lp_out = _edge_b_lp(ev_lp, eattr_lp, g2lp, E_LP,
                        W_eup1, b_eup1, W_flp, b_flp)

    return (new_scal,
            nvec16[:, :12].reshape(N_PR, NV, 3),
            pp_out,
            lp_out)


def kernel(*args):
    return _run(*args)


# speedup vs baseline: 2.8207x; 1.0108x over previous
import functools
import numpy as np
import jax
import jax.numpy as jnp
from jax import lax
from jax.experimental import pallas as pl
from jax.experimental.pallas import tpu as pltpu
from jax.experimental.pallas import tpu_sc as plsc

N_PR, N_LIG = 10000, 2000
E_PP, E_LP = 160000, 32000
NV, H = 4, 4

N_PAD = 10240
TRASH = 10000
EPP_PAD = 163840
ELP_PAD = 32768
NW = 32
CHUNK = 128


def _sel(shape, coords):
    m = np.zeros(shape, np.float32)
    for r, c in coords:
        m[r, c] = 1.0
    return m

_R_HEAD = _sel((128, 4), [(h * 32 + d, h) for h in range(4) for d in range(32)])
_P_HEAD = _sel((4, 128), [(h, h * 32 + d) for h in range(4) for d in range(32)])
_EYE4_16 = _sel((4, 16), [(i, i) for i in range(4)])
_SDEN_PP = _sel((128, 128), [(16 + h, h * 32 + d) for h in range(4) for d in range(32)])
_SDEN_LP = _sel((128, 128), [(20 + h, h * 32 + d) for h in range(4) for d in range(32)])
_P1X = _sel((128, 48), [(j * 3 + k, j * 12 + m * 3 + k) for j in range(4) for m in range(4) for k in range(3)])
_P2X = _sel((128, 48), [(m * 3 + k, j * 12 + m * 3 + k) for j in range(4) for m in range(4) for k in range(3)])
_R48 = _sel((48, 16), [(j * 12 + m * 3 + k, j * 4 + m) for j in range(4) for m in range(4) for k in range(3)])
_PL = _sel((128, 12), [(k, j * 3 + k) for j in range(4) for k in range(3)])
_PB = _sel((128, 12), [(24 + k, j * 3 + k) for j in range(4) for k in range(3)])
_PNV = _sel((128, 12), [(i, i) for i in range(12)])
_R12 = _sel((12, 4), [(j * 3 + k, j) for j in range(4) for k in range(3)])
_C12 = np.full((12, 1), 0.25, np.float32)
_G16 = _sel((16, 16), [(j, j * 3 + k) for j in range(4) for k in range(3)])
_N16 = _sel((16, 16), [(j * 3 + k, j) for j in range(4) for k in range(3)])
_K1 = _sel((16, 128), [(i, i) for i in range(12)])
_S4PP = _sel((4, 128), [(h, 16 + h) for h in range(4)])
_S4LP = _sel((4, 128), [(h, 20 + h) for h in range(4)])
_KB = _sel((16, 128), [(k, 24 + k) for k in range(3)])


def _gather(table, idx, width):
    etot = idx.shape[0]
    cpt = etot // (NW * CHUNK)
    mesh = plsc.VectorSubcoreMesh(core_axis_name="c", subcore_axis_name="s",
                                  num_cores=2)

    NB = 4
    assert cpt % NB == 0

    @functools.partial(
        pl.kernel, mesh=mesh,
        out_type=jax.ShapeDtypeStruct((etot, width), jnp.float32),
        scratch_types=[
            pltpu.VMEM((NB, CHUNK), jnp.int32),
            pltpu.VMEM((NB, CHUNK, width), jnp.float32),
        ] + [pltpu.SemaphoreType.DMA] * (3 * NB),
    )
    def k(tab_hbm, idx_hbm, out_hbm, idx_v, rows_v, *sems):
        sidx, sgat, sout = sems[:NB], sems[NB:2 * NB], sems[2 * NB:]
        wid = lax.axis_index("c") * 16 + lax.axis_index("s")
        base = wid * (cpt * CHUNK)
        pltpu.sync_copy(idx_hbm.at[pl.ds(base, CHUNK)], idx_v.at[0])

        def group(g, _):
            for t in range(NB):
                nt = (t + 1) % NB

                @pl.when(g >= 1)
                def _free_rows():
                    pltpu.make_async_copy(
                        rows_v.at[t], out_hbm.at[pl.ds(base, CHUNK)], sout[t]
                    ).wait()

                if t == 0:
                    @pl.when(g >= 1)
                    def _wait_idx0():
                        pltpu.make_async_copy(
                            idx_hbm.at[pl.ds(base, CHUNK)], idx_v.at[t],
                            sidx[t]).wait()
                else:
                    pltpu.make_async_copy(
                        idx_hbm.at[pl.ds(base, CHUNK)], idx_v.at[t],
                        sidx[t]).wait()

                pltpu.async_copy(tab_hbm.at[idx_v.at[t]], rows_v.at[t],
                                 sgat[t])

                if t == NB - 1:
                    retire = None
                else:
                    retire = g >= 1

                def _do_retire():
                    pltpu.make_async_copy(
                        tab_hbm.at[idx_v.at[nt]], rows_v.at[nt],
                        sgat[nt]).wait()
                    pltpu.async_copy(
                        rows_v.at[nt],
                        out_hbm.at[pl.ds(
                            base + (g * NB + t - (NB - 1)) * CHUNK, CHUNK)],
                        sout[nt])

                if retire is None:
                    _do_retire()
                else:
                    pl.when(retire)(_do_retire)

                if t == NB - 1:
                    @pl.when(g + 1 < cpt // NB)
                    def _next_idx():
                        pltpu.async_copy(
                            idx_hbm.at[pl.ds(base + (g * NB + t + 1) * CHUNK,
                                             CHUNK)],
                            idx_v.at[nt], sidx[nt])
                else:
                    pltpu.async_copy(
                        idx_hbm.at[pl.ds(base + (g * NB + t + 1) * CHUNK,
                                         CHUNK)],
                        idx_v.at[nt], sidx[nt])
            return _

        lax.fori_loop(0, cpt // NB, group, 0)
        for t2 in range(1, NB):
            j = cpt - NB + t2
            pltpu.make_async_copy(tab_hbm.at[idx_v.at[t2]],
                                  rows_v.at[t2], sgat[t2]).wait()
            pltpu.async_copy(rows_v.at[t2],
                             out_hbm.at[pl.ds(base + j * CHUNK, CHUNK)],
                             sout[t2])
        for t2 in range(NB):
            pltpu.make_async_copy(
                rows_v.at[t2], out_hbm.at[pl.ds(base, CHUNK)],
                sout[t2]).wait()

    return k(table, idx)


def _scatter(ev_pp, ex_pp, idx_pp, idx4_pp, ev_lp, ex_lp, idx_lp, idx4_lp,
             zeros, zeros_den):
    cpt_pp = idx_pp.shape[0] // NW
    cpt_lp = idx_lp.shape[0] // NW
    zrows = N_PAD // 16
    dlen = (N_PAD * 4) // 16
    mesh = plsc.VectorSubcoreMesh(core_axis_name="c", subcore_axis_name="s",
                                  num_cores=2)

    @functools.partial(
        pl.kernel, mesh=mesh,
        out_type=[jax.ShapeDtypeStruct((2 * N_PAD, 128), jnp.float32),
                  jax.ShapeDtypeStruct((2, N_PAD * 4), jnp.float32),
                  jax.ShapeDtypeStruct((2 * N_PAD, 128), jnp.float32),
                  jax.ShapeDtypeStruct((2, N_PAD * 4), jnp.float32)],
        scratch_types=[
            pltpu.VMEM((8, CHUNK), jnp.int32),
            pltpu.VMEM((32, CHUNK), jnp.int32),
            pltpu.VMEM((32, CHUNK), jnp.float32),
            pltpu.VMEM((2, CHUNK, 128), jnp.float32),
            pltpu.SemaphoreType.DMA,
            pltpu.VMEM_SHARED((N_PAD, 128), jnp.float32),
            pltpu.VMEM_SHARED((N_PAD * 4,), jnp.float32),
        ],
    )
    def k(evp_hbm, exp_hbm, ip_hbm, i4p_hbm, evl_hbm, exl_hbm, il_hbm, i4l_hbm,
          z_hbm, zd_hbm, out_pp, outd_pp, out_lp, outd_lp,
          idx_v, idx4_v, ex_v, ev_v, sev, acc, accd):
        c = lax.axis_index("c")
        s = lax.axis_index("s")
        GR = 8

        def phase(ev_hbm, ex_hbm, idx_hbm, idx4_hbm, out_hbm, outd_hbm, cpt, half):
            pltpu.sync_copy(z_hbm.at[pl.ds(s * zrows, zrows)],
                            acc.at[pl.ds(s * zrows, zrows)])
            pltpu.sync_copy(zd_hbm.at[pl.ds(s * dlen, dlen)],
                            accd.at[pl.ds(s * dlen, dlen)])
            plsc.subcore_barrier()
            rowbase = c * half + s * cpt
            pltpu.sync_copy(ev_hbm.at[pl.ds(rowbase * CHUNK, CHUNK)],
                            ev_v.at[0])

            def group(g, _):
                base = rowbase + g * GR
                pltpu.sync_copy(idx_hbm.at[pl.ds(base, GR)], idx_v)
                pltpu.sync_copy(idx4_hbm.at[pl.ds(base * 4, GR * 4)], idx4_v)
                pltpu.sync_copy(ex_hbm.at[pl.ds(base * 4, GR * 4)], ex_v)
                for t in range(GR):
                    j = g * GR + t
                    slot = t % 2

                    @pl.when(j >= 1)
                    def _wait_ev():
                        pltpu.make_async_copy(
                            ev_hbm.at[pl.ds(rowbase * CHUNK, CHUNK)],
                            ev_v.at[slot], sev).wait()

                    @pl.when(j + 1 < cpt)
                    def _next_ev():
                        pltpu.async_copy(
                            ev_hbm.at[pl.ds((base + t + 1) * CHUNK, CHUNK)],
                            ev_v.at[(t + 1) % 2], sev)

                    pltpu.sync_copy(ev_v.at[slot], acc.at[idx_v.at[t]],
                                    add=True)
                    for r in range(4):
                        pltpu.sync_copy(ex_v.at[t * 4 + r],
                                        accd.at[idx4_v.at[t * 4 + r]],
                                        add=True)
                return _

            lax.fori_loop(0, cpt // GR, group, 0)
            plsc.subcore_barrier()
            pltpu.sync_copy(acc.at[pl.ds(s * zrows, zrows)],
                            out_hbm.at[pl.ds(c * N_PAD + s * zrows, zrows)])
            pltpu.sync_copy(accd.at[pl.ds(s * dlen, dlen)],
                            outd_hbm.at[c].at[pl.ds(s * dlen, dlen)])
            plsc.subcore_barrier()

        phase(evp_hbm, exp_hbm, ip_hbm, i4p_hbm, out_pp, outd_pp,
              cpt_pp, cpt_pp * 16)
        phase(evl_hbm, exl_hbm, il_hbm, i4l_hbm, out_lp, outd_lp,
              cpt_lp, cpt_lp * 16)

    return k(ev_pp, ex_pp, idx_pp, idx4_pp, ev_lp, ex_lp, idx_lp, idx4_lp,
             zeros, zeros_den)


_BLK_E = 2048
_BLK_N = 512


def _full(shape):
    return pl.BlockSpec(shape, lambda i: (0, 0))


def _rows(shape):
    return pl.BlockSpec(shape, lambda i: (i, 0))


def _edge_a_body(src_ref, dst_ref, ea_ref, w1, w2, w3, v1, v2, v3, bm, af,
                 rh, ph, out_ref, ex_ref):
    src, dst, ea = src_ref[...], dst_ref[...], ea_ref[...]
    pre = src @ w1[...] + dst @ w2[...] + ea @ w3[...] + bm[...]
    h = jnp.where(pre >= 0, pre, 0.2 * pre)
    ex4 = jnp.exp((h * af[...]) @ rh[...])
    vals = src @ v1[...] + dst @ v2[...] + ea @ v3[...]
    out_ref[...] = vals * (ex4 @ ph[...])
    ex_ref[...] = ex4


def _edge_a(g1, eattr, W_msg, b_msg, a_attn, W_val):
    epad = eattr.shape[0]
    w = [W_msg[:128], W_msg[128:256], W_msg[256:384],
         W_val[:128], W_val[128:256], W_val[256:384]]
    consts = [b_msg.reshape(1, 128), a_attn.reshape(1, 128),
              jnp.asarray(_R_HEAD), jnp.asarray(_P_HEAD)]
    grid = epad // _BLK_E
    off = epad // _BLK_E
    return pl.pallas_call(
        _edge_a_body,
        grid=(grid,),
        in_specs=[_rows((_BLK_E, 128)),
                  pl.BlockSpec((_BLK_E, 128), lambda i: (i + off, 0)),
                  _rows((_BLK_E, 128))]
        + [_full((128, 128))] * 6
        + [_full((1, 128)), _full((1, 128)), _full((128, 4)), _full((4, 128))],
        out_specs=[_rows((_BLK_E, 128)), _rows((_BLK_E, 4))],
        out_shape=[jax.ShapeDtypeStruct((epad, 128), jnp.float32),
                   jax.ShapeDtypeStruct((epad, 4), jnp.float32)],
    )(g1, g1, eattr, *w, *consts)


def _node_body(prot_ref, pvec_ref, bb_ref, npp0, npp1, nlp0, nlp1,
               dpp0, dpp1, dlp0, dlp1,
               a1, a2, a3, bn1, wn2, bn2, wg, wh0, wh1,
               ph, g16, n16, k1, s4pp, s4lp, kb,
               ns_ref, nvec_ref, geom_ref):
    prot = prot_ref[...]
    dpp = dpp0[...] + dpp1[...]
    dlp = dlp0[...] + dlp1[...]
    agg0 = (npp0[...] + npp1[...]) / (dpp @ ph[...] + 1e-9)
    agg1 = (nlp0[...] + nlp1[...]) / (dlp @ ph[...] + 1e-9)
    u = jnp.maximum(prot @ a1[...] + (agg0 @ wh0[...]) @ a2[...]
                    + (agg1 @ wh1[...]) @ a3[...] + bn1[...], 0.0)
    new_scal = prot + u @ wn2[...] + bn2[...]
    g = jax.nn.sigmoid(new_scal @ wg[...])
    nvec = pvec_ref[...] * (g @ g16[...])
    n2 = (nvec * nvec) @ n16[...]
    rn = 1.0 / jnp.sqrt(jnp.maximum(n2, 1e-8))
    nv = nvec * (rn @ g16[...])
    ns_ref[...] = new_scal
    nvec_ref[...] = nvec
    geom_ref[...] = (nv @ k1[...] + dpp @ s4pp[...] + dlp @ s4lp[...]
                     + bb_ref[...] @ kb[...])


def _node(prot, pvec16, bb16, npp0, npp1, nlp0, nlp1, dpp0, dpp1, dlp0, dlp1,
          W_n1, b_n1, W_n2, b_n2, Wg16, Wh0, Wh1):
    grid = N_PAD // _BLK_N
    consts = [jnp.asarray(m) for m in
              (_P_HEAD, _G16, _N16, _K1, _S4PP, _S4LP, _KB)]
    return pl.pallas_call(
        _node_body,
        grid=(grid,),
        in_specs=[_rows((_BLK_N, 128)), _rows((_BLK_N, 16)), _rows((_BLK_N, 16))]
        + [_rows((_BLK_N, 128))] * 4
        + [_rows((_BLK_N, 4))] * 4
        + [_full((128, 128))] * 3
        + [_full((1, 128)), _full((128, 128)), _full((1, 128)),
           _full((128, 16)), _full((128, 128)), _full((128, 128))]
        + [_full((4, 128)), _full((16, 16)), _full((16, 16)),
           _full((16, 128)), _full((4, 128)), _full((4, 128)), _full((16, 128))],
        out_specs=[_rows((_BLK_N, 128)), _rows((_BLK_N, 16)), _rows((_BLK_N, 128))],
        out_shape=[jax.ShapeDtypeStruct((N_PR, 128), jnp.float32),
                   jax.ShapeDtypeStruct((N_PR, 16), jnp.float32),
                   jax.ShapeDtypeStruct((N_PAD, 128), jnp.float32)],
    )(prot, pvec16, bb16, npp0, npp1, nlp0, nlp1, dpp0, dpp1, dlp0, dlp1,
      W_n1[:128], W_n1[128:256], W_n1[256:384],
      b_n1.reshape(1, 128), W_n2, b_n2.reshape(1, 128),
      Wg16, Wh0, Wh1, *consts)


def _edge_b_pp_body(ev_ref, ea_ref, ts_ref, td_ref, e1, e2, be, wf1, wf2, bf,
                    sden, p1x, p2x, r48, out_ref):
    ev, ea, ts, td = ev_ref[...], ea_ref[...], ts_ref[...], td_ref[...]
    w = ev / (td @ sden[...] + 1e-9)
    new_pp = jnp.maximum(ea @ e1[...] + w @ e2[...] + be[...], 0.0)
    t48 = (ts @ p1x[...]) * (td @ p2x[...])
    out_ref[...] = new_pp @ wf1[...] + (t48 @ r48[...]) @ wf2[...] + bf[...]


def _edge_b_pp(ev, eattr, g2, n_out, W_eup, b_eup, W_fpp, b_fpp):
    epad = ev.shape[0]
    off = epad // _BLK_E
    consts = [jnp.asarray(m) for m in (_SDEN_PP, _P1X, _P2X, _R48)]
    return pl.pallas_call(
        _edge_b_pp_body,
        grid=(pl.cdiv(n_out, _BLK_E),),
        in_specs=[_rows((_BLK_E, 128)), _rows((_BLK_E, 128)),
                  _rows((_BLK_E, 128)),
                  pl.BlockSpec((_BLK_E, 128), lambda i: (i + off, 0)),
                  _full((128, 128)), _full((128, 128)), _full((1, 128)),
                  _full((128, 128)), _full((16, 128)), _full((1, 128)),
                  _full((128, 128)), _full((128, 48)), _full((128, 48)),
                  _full((48, 16))],
        out_specs=_rows((_BLK_E, 128)),
        out_shape=jax.ShapeDtypeStruct((n_out, 128), jnp.float32),
    )(ev, eattr, g2, g2, W_eup[:128], W_eup[128:], b_eup.reshape(1, 128),
      W_fpp[:128], W_fpp[128:], b_fpp.reshape(1, 128), *consts)


def _edge_b_lp_body(ev_ref, ea_ref, td_ref, tl_ref, e1, e2, be, wf1, wf2, bf,
                    sden, plm, pbm, pnv, r12, c12, out_ref):
    ev, ea, td, tl = ev_ref[...], ea_ref[...], td_ref[...], tl_ref[...]
    w = ev / (td @ sden[...] + 1e-9)
    new_lp = jnp.maximum(ea @ e1[...] + w @ e2[...] + be[...], 0.0)
    d12 = tl @ plm[...] - td @ pbm[...]
    n2 = (d12 * d12) @ c12[...]
    rn = 1.0 / jnp.sqrt(jnp.maximum(n2, 1e-8))
    t12 = (td @ pnv[...]) * (d12 * rn)
    out_ref[...] = new_lp @ wf1[...] + (t12 @ r12[...]) @ wf2[...] + bf[...]


def _edge_b_lp(ev, eattr, g2, n_out, W_eup, b_eup, W_flp, b_flp):
    epad = ev.shape[0]
    off = epad // _BLK_E
    consts = [jnp.asarray(m) for m in (_SDEN_LP, _PL, _PB, _PNV, _R12, _C12)]
    return pl.pallas_call(
        _edge_b_lp_body,
        grid=(pl.cdiv(n_out, _BLK_E),),
        in_specs=[_rows((_BLK_E, 128)), _rows((_BLK_E, 128)),
                  _rows((_BLK_E, 128)),
                  pl.BlockSpec((_BLK_E, 128), lambda i: (i + off, 0)),
                  _full((128, 128)), _full((128, 128)), _full((1, 128)),
                  _full((128, 128)), _full((4, 128)), _full((1, 128)),
                  _full((128, 128)), _full((128, 12)), _full((128, 12)),
                  _full((128, 12)), _full((12, 4)), _full((12, 1))],
        out_specs=_rows((_BLK_E, 128)),
        out_shape=jax.ShapeDtypeStruct((n_out, 128), jnp.float32),
    )(ev, eattr, g2, g2, W_eup[:128], W_eup[128:], b_eup.reshape(1, 128),
      W_flp[:128], W_flp[128:], b_flp.reshape(1, 128), *consts)


def _pad_rows(x, n):
    return jnp.pad(x, ((0, n - x.shape[0]),) + ((0, 0),) * (x.ndim - 1))


@jax.jit
def _run(prot_scalars, prot_vectors, lig_scalars, lig_vectors, pr_pr_eattr,
         lig_pr_eattr, pr_pr_edge_index, lig_pr_edge_index, lig_coords,
         backbone_coords, W_msg0, b_msg0, a_attn0, W_val0, W_head0, W_eup0,
         b_eup0, W_msg1, b_msg1, a_attn1, W_val1, W_head1, W_eup1, b_eup1,
         W_n1, b_n1, W_n2, b_n2, W_gate, W_flp, b_flp, W_fpp, b_fpp):
    prot = _pad_rows(prot_scalars, N_PAD)
    tab1 = jnp.concatenate([prot, lig_scalars], axis=0)
    ep, el = pr_pr_edge_index, lig_pr_edge_index
    pps = jnp.pad(ep[0], (0, EPP_PAD - E_PP))
    ppd = jnp.pad(ep[1], (0, EPP_PAD - E_PP))
    lps = jnp.pad(el[0], (0, ELP_PAD - E_LP))
    lpd = jnp.pad(el[1], (0, ELP_PAD - E_LP))
    ppd_sc = jnp.pad(ep[1], (0, EPP_PAD - E_PP), constant_values=TRASH)
    lpd_sc = jnp.pad(el[1], (0, ELP_PAD - E_LP), constant_values=TRASH)

    g1pp = _gather(tab1, jnp.concatenate([pps, ppd]), 128)
    g1lp = _gather(tab1, jnp.concatenate([lps + N_PAD, lpd]), 128)

    eattr_pp = _pad_rows(pr_pr_eattr, EPP_PAD)
    eattr_lp = _pad_rows(lig_pr_eattr, ELP_PAD)

    ev_pp, ex_pp = _edge_a(g1pp, eattr_pp, W_msg0, b_msg0, a_attn0, W_val0)
    ev_lp, ex_lp = _edge_a(g1lp, eattr_lp, W_msg1, b_msg1, a_attn1, W_val1)

    zeros = jnp.zeros((N_PAD, 128), jnp.float32)
    zeros_den = jnp.zeros((N_PAD * 4,), jnp.float32)
    idx4_pp = (ppd_sc[:, None] * 4 + jnp.arange(4, dtype=jnp.int32)[None, :])
    idx4_lp = (lpd_sc[:, None] * 4 + jnp.arange(4, dtype=jnp.int32)[None, :])
    npp, dpp, nlp, dlp = _scatter(
        ev_pp, ex_pp.reshape(-1, CHUNK),
        ppd_sc.reshape(-1, CHUNK), idx4_pp.reshape(-1, CHUNK),
        ev_lp, ex_lp.reshape(-1, CHUNK),
        lpd_sc.reshape(-1, CHUNK), idx4_lp.reshape(-1, CHUNK),
        zeros, zeros_den)
    dpp = dpp.reshape(2, N_PAD, 4)
    dlp = dlp.reshape(2, N_PAD, 4)

    pvec16 = jnp.pad(_pad_rows(prot_vectors.reshape(N_PR, 12), N_PAD),
                     ((0, 0), (0, 4)))
    bb16 = jnp.pad(_pad_rows(backbone_coords[:, 1], N_PAD), ((0, 0), (0, 13)))
    Wg16 = jnp.pad(W_gate, ((0, 0), (0, 12)))
    new_scal, nvec16, geom = _node(
        prot, pvec16, bb16, npp[:N_PAD], npp[N_PAD:], nlp[:N_PAD], nlp[N_PAD:],
        dpp[0], dpp[1], dlp[0], dlp[1],
        W_n1, b_n1, W_n2, b_n2, Wg16, W_head0, W_head1)

    ligtab = jnp.pad(lig_coords, ((0, 0), (0, 125)))
    tab2 = jnp.concatenate([geom, ligtab], axis=0)
    g2pp = _gather(tab2, jnp.concatenate([pps, ppd]), 128)
    g2lp = _gather(tab2, jnp.concatenate([lpd, lps + N_PAD]), 128)

    pp_out = _edge_b_pp(ev_pp, eattr_pp, g2pp, E_PP,
                        W_eup0, b_eup0, W_fpp, b_fpp)
    lp_out = _edge_b_lp(ev_lp, eattr_lp, g2lp, E_LP,
                        W_eup1, b_eup1, W_flp, b_flp)

    return (new_scal,
            nvec16[:, :12].reshape(N_PR, NV, 3),
            pp_out,
            lp_out)


def kernel(*args):
    return _run(*args)


# speedup vs baseline: 2.8445x; 1.0084x over previous
import functools
import numpy as np
import jax
import jax.numpy as jnp
from jax import lax
from jax.experimental import pallas as pl
from jax.experimental.pallas import tpu as pltpu
from jax.experimental.pallas import tpu_sc as plsc

N_PR, N_LIG = 10000, 2000
E_PP, E_LP = 160000, 32000
NV, H = 4, 4

N_PAD = 10240
TRASH = 10000
EPP_PAD = 163840
ELP_PAD = 32768
NW = 32
CHUNK = 128


def _sel(shape, coords):
    m = np.zeros(shape, np.float32)
    for r, c in coords:
        m[r, c] = 1.0
    return m

_R_HEAD = _sel((128, 4), [(h * 32 + d, h) for h in range(4) for d in range(32)])
_P_HEAD = _sel((4, 128), [(h, h * 32 + d) for h in range(4) for d in range(32)])
_EYE4_16 = _sel((4, 16), [(i, i) for i in range(4)])
_SDEN_PP = _sel((128, 128), [(16 + h, h * 32 + d) for h in range(4) for d in range(32)])
_SDEN_LP = _sel((128, 128), [(20 + h, h * 32 + d) for h in range(4) for d in range(32)])
_P1X = _sel((128, 48), [(j * 3 + k, j * 12 + m * 3 + k) for j in range(4) for m in range(4) for k in range(3)])
_P2X = _sel((128, 48), [(m * 3 + k, j * 12 + m * 3 + k) for j in range(4) for m in range(4) for k in range(3)])
_R48 = _sel((48, 16), [(j * 12 + m * 3 + k, j * 4 + m) for j in range(4) for m in range(4) for k in range(3)])
_PL = _sel((128, 12), [(k, j * 3 + k) for j in range(4) for k in range(3)])
_PB = _sel((128, 12), [(24 + k, j * 3 + k) for j in range(4) for k in range(3)])
_PNV = _sel((128, 12), [(i, i) for i in range(12)])
_R12 = _sel((12, 4), [(j * 3 + k, j) for j in range(4) for k in range(3)])
_C12 = np.full((12, 1), 0.25, np.float32)
_G16 = _sel((16, 16), [(j, j * 3 + k) for j in range(4) for k in range(3)])
_N16 = _sel((16, 16), [(j * 3 + k, j) for j in range(4) for k in range(3)])
_K1 = _sel((16, 128), [(i, i) for i in range(12)])
_S4PP = _sel((4, 128), [(h, 16 + h) for h in range(4)])
_S4LP = _sel((4, 128), [(h, 20 + h) for h in range(4)])
_KB = _sel((16, 128), [(k, 24 + k) for k in range(3)])


def _gather(table, idx, width):
    etot = idx.shape[0]
    cpt = etot // (NW * CHUNK)
    mesh = plsc.VectorSubcoreMesh(core_axis_name="c", subcore_axis_name="s",
                                  num_cores=2)

    NB = 4
    assert cpt % NB == 0

    @functools.partial(
        pl.kernel, mesh=mesh,
        out_type=jax.ShapeDtypeStruct((etot, width), jnp.float32),
        scratch_types=[
            pltpu.VMEM((NB, CHUNK), jnp.int32),
            pltpu.VMEM((NB, CHUNK, width), jnp.float32),
        ] + [pltpu.SemaphoreType.DMA] * (3 * NB),
    )
    def k(tab_hbm, idx_hbm, out_hbm, idx_v, rows_v, *sems):
        sidx, sgat, sout = sems[:NB], sems[NB:2 * NB], sems[2 * NB:]
        wid = lax.axis_index("c") * 16 + lax.axis_index("s")
        base = wid * (cpt * CHUNK)
        pltpu.sync_copy(idx_hbm.at[pl.ds(base, CHUNK)], idx_v.at[0])

        def group(g, _):
            for t in range(NB):
                nt = (t + 1) % NB

                @pl.when(g >= 1)
                def _free_rows():
                    pltpu.make_async_copy(
                        rows_v.at[t], out_hbm.at[pl.ds(base, CHUNK)], sout[t]
                    ).wait()

                if t == 0:
                    @pl.when(g >= 1)
                    def _wait_idx0():
                        pltpu.make_async_copy(
                            idx_hbm.at[pl.ds(base, CHUNK)], idx_v.at[t],
                            sidx[t]).wait()
                else:
                    pltpu.make_async_copy(
                        idx_hbm.at[pl.ds(base, CHUNK)], idx_v.at[t],
                        sidx[t]).wait()

                pltpu.async_copy(tab_hbm.at[idx_v.at[t]], rows_v.at[t],
                                 sgat[t])

                if t == NB - 1:
                    retire = None
                else:
                    retire = g >= 1

                def _do_retire():
                    pltpu.make_async_copy(
                        tab_hbm.at[idx_v.at[nt]], rows_v.at[nt],
                        sgat[nt]).wait()
                    pltpu.async_copy(
                        rows_v.at[nt],
                        out_hbm.at[pl.ds(
                            base + (g * NB + t - (NB - 1)) * CHUNK, CHUNK)],
                        sout[nt])

                if retire is None:
                    _do_retire()
                else:
                    pl.when(retire)(_do_retire)

                if t == NB - 1:
                    @pl.when(g + 1 < cpt // NB)
                    def _next_idx():
                        pltpu.async_copy(
                            idx_hbm.at[pl.ds(base + (g * NB + t + 1) * CHUNK,
                                             CHUNK)],
                            idx_v.at[nt], sidx[nt])
                else:
                    pltpu.async_copy(
                        idx_hbm.at[pl.ds(base + (g * NB + t + 1) * CHUNK,
                                         CHUNK)],
                        idx_v.at[nt], sidx[nt])
            return _

        lax.fori_loop(0, cpt // NB, group, 0)
        for t2 in range(1, NB):
            j = cpt - NB + t2
            pltpu.make_async_copy(tab_hbm.at[idx_v.at[t2]],
                                  rows_v.at[t2], sgat[t2]).wait()
            pltpu.async_copy(rows_v.at[t2],
                             out_hbm.at[pl.ds(base + j * CHUNK, CHUNK)],
                             sout[t2])
        for t2 in range(NB):
            pltpu.make_async_copy(
                rows_v.at[t2], out_hbm.at[pl.ds(base, CHUNK)],
                sout[t2]).wait()

    return k(table, idx)


def _scatter(ev_pp, ex_pp, idx_pp, idx4_pp, ev_lp, ex_lp, idx_lp, idx4_lp,
             zeros, zeros_den):
    cpt_pp = idx_pp.shape[0] // NW
    cpt_lp = idx_lp.shape[0] // NW
    zrows = N_PAD // 16
    dlen = (N_PAD * 4) // 16
    mesh = plsc.VectorSubcoreMesh(core_axis_name="c", subcore_axis_name="s",
                                  num_cores=2)

    @functools.partial(
        pl.kernel, mesh=mesh,
        out_type=[jax.ShapeDtypeStruct((2 * N_PAD, 128), jnp.float32),
                  jax.ShapeDtypeStruct((2, N_PAD * 4), jnp.float32),
                  jax.ShapeDtypeStruct((2 * N_PAD, 128), jnp.float32),
                  jax.ShapeDtypeStruct((2, N_PAD * 4), jnp.float32)],
        scratch_types=[
            pltpu.VMEM((8, CHUNK), jnp.int32),
            pltpu.VMEM((32, CHUNK), jnp.int32),
            pltpu.VMEM((32, CHUNK), jnp.float32),
            pltpu.VMEM((2, CHUNK, 128), jnp.float32),
            pltpu.SemaphoreType.DMA,
            pltpu.VMEM_SHARED((N_PAD, 128), jnp.float32),
            pltpu.VMEM_SHARED((N_PAD * 4,), jnp.float32),
        ],
    )
    def k(evp_hbm, exp_hbm, ip_hbm, i4p_hbm, evl_hbm, exl_hbm, il_hbm, i4l_hbm,
          z_hbm, zd_hbm, out_pp, outd_pp, out_lp, outd_lp,
          idx_v, idx4_v, ex_v, ev_v, sev, acc, accd):
        c = lax.axis_index("c")
        s = lax.axis_index("s")
        GR = 8

        def phase(ev_hbm, ex_hbm, idx_hbm, idx4_hbm, out_hbm, outd_hbm, cpt, half):
            pltpu.sync_copy(z_hbm.at[pl.ds(s * zrows, zrows)],
                            acc.at[pl.ds(s * zrows, zrows)])
            pltpu.sync_copy(zd_hbm.at[pl.ds(s * dlen, dlen)],
                            accd.at[pl.ds(s * dlen, dlen)])
            plsc.subcore_barrier()
            rowbase = c * half + s * cpt
            pltpu.sync_copy(ev_hbm.at[pl.ds(rowbase * CHUNK, CHUNK)],
                            ev_v.at[0])

            def group(g, _):
                base = rowbase + g * GR
                pltpu.sync_copy(idx_hbm.at[pl.ds(base, GR)], idx_v)
                pltpu.sync_copy(idx4_hbm.at[pl.ds(base * 4, GR * 4)], idx4_v)
                pltpu.sync_copy(ex_hbm.at[pl.ds(base * 4, GR * 4)], ex_v)
                for t in range(GR):
                    j = g * GR + t
                    slot = t % 2

                    @pl.when(j >= 1)
                    def _wait_ev():
                        pltpu.make_async_copy(
                            ev_hbm.at[pl.ds(rowbase * CHUNK, CHUNK)],
                            ev_v.at[slot], sev).wait()

                    @pl.when(j + 1 < cpt)
                    def _next_ev():
                        pltpu.async_copy(
                            ev_hbm.at[pl.ds((base + t + 1) * CHUNK, CHUNK)],
                            ev_v.at[(t + 1) % 2], sev)

                    pltpu.sync_copy(ev_v.at[slot], acc.at[idx_v.at[t]],
                                    add=True)
                    for r in range(4):
                        pltpu.sync_copy(ex_v.at[t * 4 + r],
                                        accd.at[idx4_v.at[t * 4 + r]],
                                        add=True)
                return _

            lax.fori_loop(0, cpt // GR, group, 0)
            plsc.subcore_barrier()
            pltpu.sync_copy(acc.at[pl.ds(s * zrows, zrows)],
                            out_hbm.at[pl.ds(c * N_PAD + s * zrows, zrows)])
            pltpu.sync_copy(accd.at[pl.ds(s * dlen, dlen)],
                            outd_hbm.at[c].at[pl.ds(s * dlen, dlen)])
            plsc.subcore_barrier()

        phase(evp_hbm, exp_hbm, ip_hbm, i4p_hbm, out_pp, outd_pp,
              cpt_pp, cpt_pp * 16)
        phase(evl_hbm, exl_hbm, il_hbm, i4l_hbm, out_lp, outd_lp,
              cpt_lp, cpt_lp * 16)

    return k(ev_pp, ex_pp, idx_pp, idx4_pp, ev_lp, ex_lp, idx_lp, idx4_lp,
             zeros, zeros_den)


_BLK_E = 2048
_BLK_N = 512


def _full(shape):
    return pl.BlockSpec(shape, lambda i: (0, 0))


def _rows(shape):
    return pl.BlockSpec(shape, lambda i: (i, 0))


def _edge_a_body(src_ref, dst_ref, ea_ref, w1, w2, w3, v1, v2, v3, bm, af,
                 rh, ph, out_ref, ex_ref):
    src, dst, ea = src_ref[...], dst_ref[...], ea_ref[...]
    pre = src @ w1[...] + dst @ w2[...] + ea @ w3[...] + bm[...]
    h = jnp.where(pre >= 0, pre, 0.2 * pre)
    ex4 = jnp.exp((h * af[...]) @ rh[...])
    vals = src @ v1[...] + dst @ v2[...] + ea @ v3[...]
    out_ref[...] = vals * (ex4 @ ph[...])
    ex_ref[...] = ex4


def _edge_a_alias_body(evp, exp, src_ref, dst_ref, ea_ref, w1, w2, w3, v1, v2,
                       v3, bm, af, rh, ph, out_ref, ex_ref):
    _edge_a_body(src_ref, dst_ref, ea_ref, w1, w2, w3, v1, v2, v3, bm, af,
                 rh, ph, out_ref, ex_ref)


def _edge_a(g1, eattr, W_msg, b_msg, a_attn, W_val, epad=None, boff=0,
            prev=None):
    if epad is None:
        epad = eattr.shape[0]
    nblk = g1.shape[0] // (2 * _BLK_E)
    goff = g1.shape[0] // (2 * _BLK_E)
    w = [W_msg[:128], W_msg[128:256], W_msg[256:384],
         W_val[:128], W_val[128:256], W_val[256:384]]
    consts = [b_msg.reshape(1, 128), a_attn.reshape(1, 128),
              jnp.asarray(_R_HEAD), jnp.asarray(_P_HEAD)]
    specs = [_rows((_BLK_E, 128)),
             pl.BlockSpec((_BLK_E, 128), lambda i, o=goff: (i + o, 0)),
             pl.BlockSpec((_BLK_E, 128), lambda i, o=boff: (i + o, 0))] \
        + [_full((128, 128))] * 6 \
        + [_full((1, 128)), _full((1, 128)), _full((128, 4)), _full((4, 128))]
    out_specs = [pl.BlockSpec((_BLK_E, 128), lambda i, o=boff: (i + o, 0)),
                 pl.BlockSpec((_BLK_E, 4), lambda i, o=boff: (i + o, 0))]
    out_shape = [jax.ShapeDtypeStruct((epad, 128), jnp.float32),
                 jax.ShapeDtypeStruct((epad, 4), jnp.float32)]
    if prev is None:
        return pl.pallas_call(
            _edge_a_body, grid=(nblk,), in_specs=specs,
            out_specs=out_specs, out_shape=out_shape,
        )(g1, g1, eattr, *w, *consts)
    any_spec = pl.BlockSpec(memory_space=pl.ANY)
    return pl.pallas_call(
        _edge_a_alias_body, grid=(nblk,),
        in_specs=[any_spec, any_spec] + specs,
        out_specs=out_specs, out_shape=out_shape,
        input_output_aliases={0: 0, 1: 1},
    )(prev[0], prev[1], g1, g1, eattr, *w, *consts)


def _node_body(prot_ref, pvec_ref, bb_ref, npp0, npp1, nlp0, nlp1,
               dpp0, dpp1, dlp0, dlp1,
               a1, a2, a3, bn1, wn2, bn2, wg, wh0, wh1,
               ph, g16, n16, k1, s4pp, s4lp, kb,
               ns_ref, nvec_ref, geom_ref):
    prot = prot_ref[...]
    dpp = dpp0[...] + dpp1[...]
    dlp = dlp0[...] + dlp1[...]
    agg0 = (npp0[...] + npp1[...]) / (dpp @ ph[...] + 1e-9)
    agg1 = (nlp0[...] + nlp1[...]) / (dlp @ ph[...] + 1e-9)
    u = jnp.maximum(prot @ a1[...] + (agg0 @ wh0[...]) @ a2[...]
                    + (agg1 @ wh1[...]) @ a3[...] + bn1[...], 0.0)
    new_scal = prot + u @ wn2[...] + bn2[...]
    g = jax.nn.sigmoid(new_scal @ wg[...])
    nvec = pvec_ref[...] * (g @ g16[...])
    n2 = (nvec * nvec) @ n16[...]
    rn = 1.0 / jnp.sqrt(jnp.maximum(n2, 1e-8))
    nv = nvec * (rn @ g16[...])
    ns_ref[...] = new_scal
    nvec_ref[...] = nvec
    geom_ref[...] = (nv @ k1[...] + dpp @ s4pp[...] + dlp @ s4lp[...]
                     + bb_ref[...] @ kb[...])


def _node(prot, pvec16, bb16, npp0, npp1, nlp0, nlp1, dpp0, dpp1, dlp0, dlp1,
          W_n1, b_n1, W_n2, b_n2, Wg16, Wh0, Wh1):
    grid = N_PAD // _BLK_N
    consts = [jnp.asarray(m) for m in
              (_P_HEAD, _G16, _N16, _K1, _S4PP, _S4LP, _KB)]
    return pl.pallas_call(
        _node_body,
        grid=(grid,),
        in_specs=[_rows((_BLK_N, 128)), _rows((_BLK_N, 16)), _rows((_BLK_N, 16))]
        + [_rows((_BLK_N, 128))] * 4
        + [_rows((_BLK_N, 4))] * 4
        + [_full((128, 128))] * 3
        + [_full((1, 128)), _full((128, 128)), _full((1, 128)),
           _full((128, 16)), _full((128, 128)), _full((128, 128))]
        + [_full((4, 128)), _full((16, 16)), _full((16, 16)),
           _full((16, 128)), _full((4, 128)), _full((4, 128)), _full((16, 128))],
        out_specs=[_rows((_BLK_N, 128)), _rows((_BLK_N, 16)), _rows((_BLK_N, 128))],
        out_shape=[jax.ShapeDtypeStruct((N_PR, 128), jnp.float32),
                   jax.ShapeDtypeStruct((N_PR, 16), jnp.float32),
                   jax.ShapeDtypeStruct((N_PAD, 128), jnp.float32)],
    )(prot, pvec16, bb16, npp0, npp1, nlp0, nlp1, dpp0, dpp1, dlp0, dlp1,
      W_n1[:128], W_n1[128:256], W_n1[256:384],
      b_n1.reshape(1, 128), W_n2, b_n2.reshape(1, 128),
      Wg16, Wh0, Wh1, *consts)


def _edge_b_pp_body(ev_ref, ea_ref, ts_ref, td_ref, e1, e2, be, wf1, wf2, bf,
                    sden, p1x, p2x, r48, out_ref):
    ev, ea, ts, td = ev_ref[...], ea_ref[...], ts_ref[...], td_ref[...]
    w = ev / (td @ sden[...] + 1e-9)
    new_pp = jnp.maximum(ea @ e1[...] + w @ e2[...] + be[...], 0.0)
    t48 = (ts @ p1x[...]) * (td @ p2x[...])
    out_ref[...] = new_pp @ wf1[...] + (t48 @ r48[...]) @ wf2[...] + bf[...]


def _edge_b_pp_alias_body(pv, ev_ref, ea_ref, ts_ref, td_ref, e1, e2, be, wf1,
                          wf2, bf, sden, p1x, p2x, r48, out_ref):
    _edge_b_pp_body(ev_ref, ea_ref, ts_ref, td_ref, e1, e2, be, wf1, wf2, bf,
                    sden, p1x, p2x, r48, out_ref)


def _edge_b_pp(ev, eattr, g2, n_out, W_eup, b_eup, W_fpp, b_fpp,
               nblk=None, boff=0, prev=None):
    goff = g2.shape[0] // (2 * _BLK_E)
    if nblk is None:
        nblk = pl.cdiv(n_out, _BLK_E)
    consts = [jnp.asarray(m) for m in (_SDEN_PP, _P1X, _P2X, _R48)]
    specs = [pl.BlockSpec((_BLK_E, 128), lambda i, o=boff: (i + o, 0)),
             pl.BlockSpec((_BLK_E, 128), lambda i, o=boff: (i + o, 0)),
             _rows((_BLK_E, 128)),
             pl.BlockSpec((_BLK_E, 128), lambda i, o=goff: (i + o, 0)),
             _full((128, 128)), _full((128, 128)), _full((1, 128)),
             _full((128, 128)), _full((16, 128)), _full((1, 128)),
             _full((128, 128)), _full((128, 48)), _full((128, 48)),
             _full((48, 16))]
    out_spec = pl.BlockSpec((_BLK_E, 128), lambda i, o=boff: (i + o, 0))
    out_shape = jax.ShapeDtypeStruct((n_out, 128), jnp.float32)
    args = (ev, eattr, g2, g2, W_eup[:128], W_eup[128:],
            b_eup.reshape(1, 128), W_fpp[:128], W_fpp[128:],
            b_fpp.reshape(1, 128), *consts)
    if prev is None:
        return pl.pallas_call(
            _edge_b_pp_body, grid=(nblk,), in_specs=specs,
            out_specs=out_spec, out_shape=out_shape)(*args)
    any_spec = pl.BlockSpec(memory_space=pl.ANY)
    return pl.pallas_call(
        _edge_b_pp_alias_body, grid=(nblk,), in_specs=[any_spec] + specs,
        out_specs=out_spec, out_shape=out_shape,
        input_output_aliases={0: 0})(prev, *args)


def _edge_b_lp_body(ev_ref, ea_ref, td_ref, tl_ref, e1, e2, be, wf1, wf2, bf,
                    sden, plm, pbm, pnv, r12, c12, out_ref):
    ev, ea, td, tl = ev_ref[...], ea_ref[...], td_ref[...], tl_ref[...]
    w = ev / (td @ sden[...] + 1e-9)
    new_lp = jnp.maximum(ea @ e1[...] + w @ e2[...] + be[...], 0.0)
    d12 = tl @ plm[...] - td @ pbm[...]
    n2 = (d12 * d12) @ c12[...]
    rn = 1.0 / jnp.sqrt(jnp.maximum(n2, 1e-8))
    t12 = (td @ pnv[...]) * (d12 * rn)
    out_ref[...] = new_lp @ wf1[...] + (t12 @ r12[...]) @ wf2[...] + bf[...]


def _edge_b_lp(ev, eattr, g2, n_out, W_eup, b_eup, W_flp, b_flp):
    epad = ev.shape[0]
    off = epad // _BLK_E
    consts = [jnp.asarray(m) for m in (_SDEN_LP, _PL, _PB, _PNV, _R12, _C12)]
    return pl.pallas_call(
        _edge_b_lp_body,
        grid=(pl.cdiv(n_out, _BLK_E),),
        in_specs=[_rows((_BLK_E, 128)), _rows((_BLK_E, 128)),
                  _rows((_BLK_E, 128)),
                  pl.BlockSpec((_BLK_E, 128), lambda i: (i + off, 0)),
                  _full((128, 128)), _full((128, 128)), _full((1, 128)),
                  _full((128, 128)), _full((4, 128)), _full((1, 128)),
                  _full((128, 128)), _full((128, 12)), _full((128, 12)),
                  _full((128, 12)), _full((12, 4)), _full((12, 1))],
        out_specs=_rows((_BLK_E, 128)),
        out_shape=jax.ShapeDtypeStruct((n_out, 128), jnp.float32),
    )(ev, eattr, g2, g2, W_eup[:128], W_eup[128:], b_eup.reshape(1, 128),
      W_flp[:128], W_flp[128:], b_flp.reshape(1, 128), *consts)


def _pad_rows(x, n):
    return jnp.pad(x, ((0, n - x.shape[0]),) + ((0, 0),) * (x.ndim - 1))


@jax.jit
def _run(prot_scalars, prot_vectors, lig_scalars, lig_vectors, pr_pr_eattr,
         lig_pr_eattr, pr_pr_edge_index, lig_pr_edge_index, lig_coords,
         backbone_coords, W_msg0, b_msg0, a_attn0, W_val0, W_head0, W_eup0,
         b_eup0, W_msg1, b_msg1, a_attn1, W_val1, W_head1, W_eup1, b_eup1,
         W_n1, b_n1, W_n2, b_n2, W_gate, W_flp, b_flp, W_fpp, b_fpp):
    prot = _pad_rows(prot_scalars, N_PAD)
    tab1 = jnp.concatenate([prot, lig_scalars], axis=0)
    ep, el = pr_pr_edge_index, lig_pr_edge_index
    pps = jnp.pad(ep[0], (0, EPP_PAD - E_PP))
    ppd = jnp.pad(ep[1], (0, EPP_PAD - E_PP))
    lps = jnp.pad(el[0], (0, ELP_PAD - E_LP))
    lpd = jnp.pad(el[1], (0, ELP_PAD - E_LP))
    ppd_sc = jnp.pad(ep[1], (0, EPP_PAD - E_PP), constant_values=TRASH)
    lpd_sc = jnp.pad(el[1], (0, ELP_PAD - E_LP), constant_values=TRASH)

    half = EPP_PAD // 2
    hb = half // _BLK_E
    g1ppA = _gather(tab1, jnp.concatenate([pps[:half], ppd[:half]]), 128)
    g1ppB = _gather(tab1, jnp.concatenate([pps[half:], ppd[half:]]), 128)
    g1lp = _gather(tab1, jnp.concatenate([lps + N_PAD, lpd]), 128)

    eattr_pp = _pad_rows(pr_pr_eattr, EPP_PAD)
    eattr_lp = _pad_rows(lig_pr_eattr, ELP_PAD)

    evA, exA = _edge_a(g1ppA, eattr_pp, W_msg0, b_msg0, a_attn0, W_val0,
                       epad=EPP_PAD)
    ev_pp, ex_pp = _edge_a(g1ppB, eattr_pp, W_msg0, b_msg0, a_attn0, W_val0,
                           epad=EPP_PAD, boff=hb, prev=(evA, exA))
    ev_lp, ex_lp = _edge_a(g1lp, eattr_lp, W_msg1, b_msg1, a_attn1, W_val1)

    zeros = jnp.zeros((N_PAD, 128), jnp.float32)
    zeros_den = jnp.zeros((N_PAD * 4,), jnp.float32)
    idx4_pp = (ppd_sc[:, None] * 4 + jnp.arange(4, dtype=jnp.int32)[None, :])
    idx4_lp = (lpd_sc[:, None] * 4 + jnp.arange(4, dtype=jnp.int32)[None, :])
    npp, dpp, nlp, dlp = _scatter(
        ev_pp, ex_pp.reshape(-1, CHUNK),
        ppd_sc.reshape(-1, CHUNK), idx4_pp.reshape(-1, CHUNK),
        ev_lp, ex_lp.reshape(-1, CHUNK),
        lpd_sc.reshape(-1, CHUNK), idx4_lp.reshape(-1, CHUNK),
        zeros, zeros_den)
    dpp = dpp.reshape(2, N_PAD, 4)
    dlp = dlp.reshape(2, N_PAD, 4)

    pvec16 = jnp.pad(_pad_rows(prot_vectors.reshape(N_PR, 12), N_PAD),
                     ((0, 0), (0, 4)))
    bb16 = jnp.pad(_pad_rows(backbone_coords[:, 1], N_PAD), ((0, 0), (0, 13)))
    Wg16 = jnp.pad(W_gate, ((0, 0), (0, 12)))
    new_scal, nvec16, geom = _node(
        prot, pvec16, bb16, npp[:N_PAD], npp[N_PAD:], nlp[:N_PAD], nlp[N_PAD:],
        dpp[0], dpp[1], dlp[0], dlp[1],
        W_n1, b_n1, W_n2, b_n2, Wg16, W_head0, W_head1)

    ligtab = jnp.pad(lig_coords, ((0, 0), (0, 125)))
    tab2 = jnp.concatenate([geom, ligtab], axis=0)
    g2ppA = _gather(tab2, jnp.concatenate([pps[:half], ppd[:half]]), 128)
    g2ppB = _gather(tab2, jnp.concatenate([pps[half:], ppd[half:]]), 128)
    g2lp = _gather(tab2, jnp.concatenate([lpd, lps + N_PAD]), 128)

    ppoA = _edge_b_pp(ev_pp, eattr_pp, g2ppA, E_PP,
                      W_eup0, b_eup0, W_fpp, b_fpp, nblk=hb)
    pp_out = _edge_b_pp(ev_pp, eattr_pp, g2ppB, E_PP,
                        W_eup0, b_eup0, W_fpp, b_fpp,
                        nblk=pl.cdiv(E_PP, _BLK_E) - hb, boff=hb, prev=ppoA)
    lp_out = _edge_b_lp(ev_lp, eattr_lp, g2lp, E_LP,
                        W_eup1, b_eup1, W_flp, b_flp)

    return (new_scal,
            nvec16[:, :12].reshape(N_PR, NV, 3),
            pp_out,
            lp_out)


def kernel(*args):
    return _run(*args)


# speedup vs baseline: 3.2072x; 1.1275x over previous
import functools
import numpy as np
import jax
import jax.numpy as jnp
from jax import lax
from jax.experimental import pallas as pl
from jax.experimental.pallas import tpu as pltpu
from jax.experimental.pallas import tpu_sc as plsc

N_PR, N_LIG = 10000, 2000
E_PP, E_LP = 160000, 32000
NV, H = 4, 4

N_PAD = 10240
TRASH = 10000
EPP_PAD = 163840
ELP_PAD = 32768
NW = 32
CHUNK = 128


def _sel(shape, coords):
    m = np.zeros(shape, np.float32)
    for r, c in coords:
        m[r, c] = 1.0
    return m

_R_HEAD = _sel((128, 4), [(h * 32 + d, h) for h in range(4) for d in range(32)])
_P_HEAD = _sel((4, 128), [(h, h * 32 + d) for h in range(4) for d in range(32)])
_EYE4_16 = _sel((4, 16), [(i, i) for i in range(4)])
_SDEN_PP = _sel((32, 128), [(16 + h, h * 32 + d) for h in range(4) for d in range(32)])
_SDEN_LP = _sel((32, 128), [(20 + h, h * 32 + d) for h in range(4) for d in range(32)])
_P1X = _sel((32, 48), [(j * 3 + k, j * 12 + m * 3 + k) for j in range(4) for m in range(4) for k in range(3)])
_P2X = _sel((32, 48), [(m * 3 + k, j * 12 + m * 3 + k) for j in range(4) for m in range(4) for k in range(3)])
_R48 = _sel((48, 16), [(j * 12 + m * 3 + k, j * 4 + m) for j in range(4) for m in range(4) for k in range(3)])
_PL = _sel((32, 12), [(k, j * 3 + k) for j in range(4) for k in range(3)])
_PB = _sel((32, 12), [(24 + k, j * 3 + k) for j in range(4) for k in range(3)])
_PNV = _sel((32, 12), [(i, i) for i in range(12)])
_R12 = _sel((12, 4), [(j * 3 + k, j) for j in range(4) for k in range(3)])
_C12 = np.full((12, 1), 0.25, np.float32)
_G16 = _sel((16, 16), [(j, j * 3 + k) for j in range(4) for k in range(3)])
_N16 = _sel((16, 16), [(j * 3 + k, j) for j in range(4) for k in range(3)])
_K1 = _sel((16, 32), [(i, i) for i in range(12)])
_S4PP = _sel((4, 32), [(h, 16 + h) for h in range(4)])
_S4LP = _sel((4, 32), [(h, 20 + h) for h in range(4)])
_KB = _sel((16, 32), [(k, 24 + k) for k in range(3)])


def _gather(table, idx, width):
    etot = idx.shape[0]
    cpt = etot // (NW * CHUNK)
    mesh = plsc.VectorSubcoreMesh(core_axis_name="c", subcore_axis_name="s",
                                  num_cores=2)

    NB = 4
    assert cpt % NB == 0

    kw = {}
    if width != 128:
        kw["compiler_params"] = pltpu.CompilerParams(use_tc_tiling_on_sc=False)

    @functools.partial(
        pl.kernel, mesh=mesh,
        out_type=jax.ShapeDtypeStruct((etot, width), jnp.float32),
        scratch_types=[
            pltpu.VMEM((NB, CHUNK), jnp.int32),
            pltpu.VMEM((NB, CHUNK, width), jnp.float32),
        ] + [pltpu.SemaphoreType.DMA] * (3 * NB),
        **kw,
    )
    def k(tab_hbm, idx_hbm, out_hbm, idx_v, rows_v, *sems):
        sidx, sgat, sout = sems[:NB], sems[NB:2 * NB], sems[2 * NB:]
        wid = lax.axis_index("c") * 16 + lax.axis_index("s")
        base = wid * (cpt * CHUNK)
        pltpu.sync_copy(idx_hbm.at[pl.ds(base, CHUNK)], idx_v.at[0])

        def group(g, _):
            for t in range(NB):
                nt = (t + 1) % NB

                @pl.when(g >= 1)
                def _free_rows():
                    pltpu.make_async_copy(
                        rows_v.at[t], out_hbm.at[pl.ds(base, CHUNK)], sout[t]
                    ).wait()

                if t == 0:
                    @pl.when(g >= 1)
                    def _wait_idx0():
                        pltpu.make_async_copy(
                            idx_hbm.at[pl.ds(base, CHUNK)], idx_v.at[t],
                            sidx[t]).wait()
                else:
                    pltpu.make_async_copy(
                        idx_hbm.at[pl.ds(base, CHUNK)], idx_v.at[t],
                        sidx[t]).wait()

                pltpu.async_copy(tab_hbm.at[idx_v.at[t]], rows_v.at[t],
                                 sgat[t])

                if t == NB - 1:
                    retire = None
                else:
                    retire = g >= 1

                def _do_retire():
                    pltpu.make_async_copy(
                        tab_hbm.at[idx_v.at[nt]], rows_v.at[nt],
                        sgat[nt]).wait()
                    pltpu.async_copy(
                        rows_v.at[nt],
                        out_hbm.at[pl.ds(
                            base + (g * NB + t - (NB - 1)) * CHUNK, CHUNK)],
                        sout[nt])

                if retire is None:
                    _do_retire()
                else:
                    pl.when(retire)(_do_retire)

                if t == NB - 1:
                    @pl.when(g + 1 < cpt // NB)
                    def _next_idx():
                        pltpu.async_copy(
                            idx_hbm.at[pl.ds(base + (g * NB + t + 1) * CHUNK,
                                             CHUNK)],
                            idx_v.at[nt], sidx[nt])
                else:
                    pltpu.async_copy(
                        idx_hbm.at[pl.ds(base + (g * NB + t + 1) * CHUNK,
                                         CHUNK)],
                        idx_v.at[nt], sidx[nt])
            return _

        lax.fori_loop(0, cpt // NB, group, 0)
        for t2 in range(1, NB):
            j = cpt - NB + t2
            pltpu.make_async_copy(tab_hbm.at[idx_v.at[t2]],
                                  rows_v.at[t2], sgat[t2]).wait()
            pltpu.async_copy(rows_v.at[t2],
                             out_hbm.at[pl.ds(base + j * CHUNK, CHUNK)],
                             sout[t2])
        for t2 in range(NB):
            pltpu.make_async_copy(
                rows_v.at[t2], out_hbm.at[pl.ds(base, CHUNK)],
                sout[t2]).wait()

    return k(table, idx)


def _scatter(ev_pp, ex_pp, idx_pp, idx4_pp, ev_lp, ex_lp, idx_lp, idx4_lp,
             zeros, zeros_den):
    cpt_pp = idx_pp.shape[0] // NW
    cpt_lp = idx_lp.shape[0] // NW
    zrows = N_PAD // 16
    dlen = (N_PAD * 4) // 16
    mesh = plsc.VectorSubcoreMesh(core_axis_name="c", subcore_axis_name="s",
                                  num_cores=2)

    @functools.partial(
        pl.kernel, mesh=mesh,
        out_type=[jax.ShapeDtypeStruct((2 * N_PAD, 128), jnp.float32),
                  jax.ShapeDtypeStruct((2, N_PAD * 4), jnp.float32),
                  jax.ShapeDtypeStruct((2 * N_PAD, 128), jnp.float32),
                  jax.ShapeDtypeStruct((2, N_PAD * 4), jnp.float32)],
        scratch_types=[
            pltpu.VMEM((8, CHUNK), jnp.int32),
            pltpu.VMEM((32, CHUNK), jnp.int32),
            pltpu.VMEM((32, CHUNK), jnp.float32),
            pltpu.VMEM((2, CHUNK, 128), jnp.float32),
            pltpu.SemaphoreType.DMA,
            pltpu.VMEM_SHARED((N_PAD, 128), jnp.float32),
            pltpu.VMEM_SHARED((N_PAD * 4,), jnp.float32),
        ],
    )
    def k(evp_hbm, exp_hbm, ip_hbm, i4p_hbm, evl_hbm, exl_hbm, il_hbm, i4l_hbm,
          z_hbm, zd_hbm, out_pp, outd_pp, out_lp, outd_lp,
          idx_v, idx4_v, ex_v, ev_v, sev, acc, accd):
        c = lax.axis_index("c")
        s = lax.axis_index("s")
        GR = 8

        def phase(ev_hbm, ex_hbm, idx_hbm, idx4_hbm, out_hbm, outd_hbm, cpt, half):
            pltpu.sync_copy(z_hbm.at[pl.ds(s * zrows, zrows)],
                            acc.at[pl.ds(s * zrows, zrows)])
            pltpu.sync_copy(zd_hbm.at[pl.ds(s * dlen, dlen)],
                            accd.at[pl.ds(s * dlen, dlen)])
            plsc.subcore_barrier()
            rowbase = c * half + s * cpt
            pltpu.sync_copy(ev_hbm.at[pl.ds(rowbase * CHUNK, CHUNK)],
                            ev_v.at[0])

            def group(g, _):
                base = rowbase + g * GR
                pltpu.sync_copy(idx_hbm.at[pl.ds(base, GR)], idx_v)
                pltpu.sync_copy(idx4_hbm.at[pl.ds(base * 4, GR * 4)], idx4_v)
                pltpu.sync_copy(ex_hbm.at[pl.ds(base * 4, GR * 4)], ex_v)
                for t in range(GR):
                    j = g * GR + t
                    slot = t % 2

                    @pl.when(j >= 1)
                    def _wait_ev():
                        pltpu.make_async_copy(
                            ev_hbm.at[pl.ds(rowbase * CHUNK, CHUNK)],
                            ev_v.at[slot], sev).wait()

                    @pl.when(j + 1 < cpt)
                    def _next_ev():
                        pltpu.async_copy(
                            ev_hbm.at[pl.ds((base + t + 1) * CHUNK, CHUNK)],
                            ev_v.at[(t + 1) % 2], sev)

                    pltpu.sync_copy(ev_v.at[slot], acc.at[idx_v.at[t]],
                                    add=True)
                    for r in range(4):
                        pltpu.sync_copy(ex_v.at[t * 4 + r],
                                        accd.at[idx4_v.at[t * 4 + r]],
                                        add=True)
                return _

            lax.fori_loop(0, cpt // GR, group, 0)
            plsc.subcore_barrier()
            pltpu.sync_copy(acc.at[pl.ds(s * zrows, zrows)],
                            out_hbm.at[pl.ds(c * N_PAD + s * zrows, zrows)])
            pltpu.sync_copy(accd.at[pl.ds(s * dlen, dlen)],
                            outd_hbm.at[c].at[pl.ds(s * dlen, dlen)])
            plsc.subcore_barrier()

        phase(evp_hbm, exp_hbm, ip_hbm, i4p_hbm, out_pp, outd_pp,
              cpt_pp, cpt_pp * 16)
        phase(evl_hbm, exl_hbm, il_hbm, i4l_hbm, out_lp, outd_lp,
              cpt_lp, cpt_lp * 16)

    return k(ev_pp, ex_pp, idx_pp, idx4_pp, ev_lp, ex_lp, idx_lp, idx4_lp,
             zeros, zeros_den)


_BLK_E = 2048
_BLK_N = 512


def _full(shape):
    return pl.BlockSpec(shape, lambda i: (0, 0))


def _rows(shape):
    return pl.BlockSpec(shape, lambda i: (i, 0))


def _edge_a_body(src_ref, dst_ref, ea_ref, w1, w2, w3, v1, v2, v3, bm, af,
                 rh, ph, out_ref, ex_ref):
    src, dst, ea = src_ref[...], dst_ref[...], ea_ref[...]
    pre = src @ w1[...] + dst @ w2[...] + ea @ w3[...] + bm[...]
    h = jnp.where(pre >= 0, pre, 0.2 * pre)
    ex4 = jnp.exp((h * af[...]) @ rh[...])
    vals = src @ v1[...] + dst @ v2[...] + ea @ v3[...]
    out_ref[...] = vals * (ex4 @ ph[...])
    ex_ref[...] = ex4


def _edge_a_alias_body(evp, exp, src_ref, dst_ref, ea_ref, w1, w2, w3, v1, v2,
                       v3, bm, af, rh, ph, out_ref, ex_ref):
    _edge_a_body(src_ref, dst_ref, ea_ref, w1, w2, w3, v1, v2, v3, bm, af,
                 rh, ph, out_ref, ex_ref)


def _edge_a(g1, eattr, W_msg, b_msg, a_attn, W_val, epad=None, boff=0,
            prev=None):
    if epad is None:
        epad = eattr.shape[0]
    nblk = g1.shape[0] // (2 * _BLK_E)
    goff = g1.shape[0] // (2 * _BLK_E)
    w = [W_msg[:128], W_msg[128:256], W_msg[256:384],
         W_val[:128], W_val[128:256], W_val[256:384]]
    consts = [b_msg.reshape(1, 128), a_attn.reshape(1, 128),
              jnp.asarray(_R_HEAD), jnp.asarray(_P_HEAD)]
    specs = [_rows((_BLK_E, 128)),
             pl.BlockSpec((_BLK_E, 128), lambda i, o=goff: (i + o, 0)),
             pl.BlockSpec((_BLK_E, 128), lambda i, o=boff: (i + o, 0))] \
        + [_full((128, 128))] * 6 \
        + [_full((1, 128)), _full((1, 128)), _full((128, 4)), _full((4, 128))]
    out_specs = [pl.BlockSpec((_BLK_E, 128), lambda i, o=boff: (i + o, 0)),
                 pl.BlockSpec((_BLK_E, 4), lambda i, o=boff: (i + o, 0))]
    out_shape = [jax.ShapeDtypeStruct((epad, 128), jnp.float32),
                 jax.ShapeDtypeStruct((epad, 4), jnp.float32)]
    if prev is None:
        return pl.pallas_call(
            _edge_a_body, grid=(nblk,), in_specs=specs,
            out_specs=out_specs, out_shape=out_shape,
        )(g1, g1, eattr, *w, *consts)
    any_spec = pl.BlockSpec(memory_space=pl.ANY)
    return pl.pallas_call(
        _edge_a_alias_body, grid=(nblk,),
        in_specs=[any_spec, any_spec] + specs,
        out_specs=out_specs, out_shape=out_shape,
        input_output_aliases={0: 0, 1: 1},
    )(prev[0], prev[1], g1, g1, eattr, *w, *consts)


def _node_body(prot_ref, pvec_ref, bb_ref, npp0, npp1, nlp0, nlp1,
               dpp0, dpp1, dlp0, dlp1,
               a1, a2, a3, bn1, wn2, bn2, wg, wh0, wh1,
               ph, g16, n16, k1, s4pp, s4lp, kb,
               ns_ref, nvec_ref, geom_ref):
    prot = prot_ref[...]
    dpp = dpp0[...] + dpp1[...]
    dlp = dlp0[...] + dlp1[...]
    agg0 = (npp0[...] + npp1[...]) / (dpp @ ph[...] + 1e-9)
    agg1 = (nlp0[...] + nlp1[...]) / (dlp @ ph[...] + 1e-9)
    u = jnp.maximum(prot @ a1[...] + (agg0 @ wh0[...]) @ a2[...]
                    + (agg1 @ wh1[...]) @ a3[...] + bn1[...], 0.0)
    new_scal = prot + u @ wn2[...] + bn2[...]
    g = jax.nn.sigmoid(new_scal @ wg[...])
    nvec = pvec_ref[...] * (g @ g16[...])
    n2 = (nvec * nvec) @ n16[...]
    rn = 1.0 / jnp.sqrt(jnp.maximum(n2, 1e-8))
    nv = nvec * (rn @ g16[...])
    ns_ref[...] = new_scal
    nvec_ref[...] = nvec
    geom_ref[...] = (nv @ k1[...] + dpp @ s4pp[...] + dlp @ s4lp[...]
                     + bb_ref[...] @ kb[...])


def _node(prot, pvec16, bb16, npp0, npp1, nlp0, nlp1, dpp0, dpp1, dlp0, dlp1,
          W_n1, b_n1, W_n2, b_n2, Wg16, Wh0, Wh1):
    grid = N_PAD // _BLK_N
    consts = [jnp.asarray(m) for m in
              (_P_HEAD, _G16, _N16, _K1, _S4PP, _S4LP, _KB)]
    return pl.pallas_call(
        _node_body,
        grid=(grid,),
        in_specs=[_rows((_BLK_N, 128)), _rows((_BLK_N, 16)), _rows((_BLK_N, 16))]
        + [_rows((_BLK_N, 128))] * 4
        + [_rows((_BLK_N, 4))] * 4
        + [_full((128, 128))] * 3
        + [_full((1, 128)), _full((128, 128)), _full((1, 128)),
           _full((128, 16)), _full((128, 128)), _full((128, 128))]
        + [_full((4, 128)), _full((16, 16)), _full((16, 16)),
           _full((16, 32)), _full((4, 32)), _full((4, 32)), _full((16, 32))],
        out_specs=[_rows((_BLK_N, 128)), _rows((_BLK_N, 16)), _rows((_BLK_N, 32))],
        out_shape=[jax.ShapeDtypeStruct((N_PR, 128), jnp.float32),
                   jax.ShapeDtypeStruct((N_PR, 16), jnp.float32),
                   jax.ShapeDtypeStruct((N_PAD, 32), jnp.float32)],
    )(prot, pvec16, bb16, npp0, npp1, nlp0, nlp1, dpp0, dpp1, dlp0, dlp1,
      W_n1[:128], W_n1[128:256], W_n1[256:384],
      b_n1.reshape(1, 128), W_n2, b_n2.reshape(1, 128),
      Wg16, Wh0, Wh1, *consts)


def _edge_b_pp_body(ev_ref, ea_ref, ts_ref, td_ref, e1, e2, be, wf1, wf2, bf,
                    sden, p1x, p2x, r48, out_ref):
    ev, ea, ts, td = ev_ref[...], ea_ref[...], ts_ref[...], td_ref[...]
    w = ev / (td @ sden[...] + 1e-9)
    new_pp = jnp.maximum(ea @ e1[...] + w @ e2[...] + be[...], 0.0)
    t48 = (ts @ p1x[...]) * (td @ p2x[...])
    out_ref[...] = new_pp @ wf1[...] + (t48 @ r48[...]) @ wf2[...] + bf[...]


def _edge_b_pp_alias_body(pv, ev_ref, ea_ref, ts_ref, td_ref, e1, e2, be, wf1,
                          wf2, bf, sden, p1x, p2x, r48, out_ref):
    _edge_b_pp_body(ev_ref, ea_ref, ts_ref, td_ref, e1, e2, be, wf1, wf2, bf,
                    sden, p1x, p2x, r48, out_ref)


def _edge_b_pp(ev, eattr, g2, n_out, W_eup, b_eup, W_fpp, b_fpp,
               nblk=None, boff=0, prev=None):
    goff = g2.shape[0] // (2 * _BLK_E)
    if nblk is None:
        nblk = pl.cdiv(n_out, _BLK_E)
    consts = [jnp.asarray(m) for m in (_SDEN_PP, _P1X, _P2X, _R48)]
    specs = [pl.BlockSpec((_BLK_E, 128), lambda i, o=boff: (i + o, 0)),
             pl.BlockSpec((_BLK_E, 128), lambda i, o=boff: (i + o, 0)),
             _rows((_BLK_E, 32)),
             pl.BlockSpec((_BLK_E, 32), lambda i, o=goff: (i + o, 0)),
             _full((128, 128)), _full((128, 128)), _full((1, 128)),
             _full((128, 128)), _full((16, 128)), _full((1, 128)),
             _full((32, 128)), _full((32, 48)), _full((32, 48)),
             _full((48, 16))]
    out_spec = pl.BlockSpec((_BLK_E, 128), lambda i, o=boff: (i + o, 0))
    out_shape = jax.ShapeDtypeStruct((n_out, 128), jnp.float32)
    args = (ev, eattr, g2, g2, W_eup[:128], W_eup[128:],
            b_eup.reshape(1, 128), W_fpp[:128], W_fpp[128:],
            b_fpp.reshape(1, 128), *consts)
    if prev is None:
        return pl.pallas_call(
            _edge_b_pp_body, grid=(nblk,), in_specs=specs,
            out_specs=out_spec, out_shape=out_shape)(*args)
    any_spec = pl.BlockSpec(memory_space=pl.ANY)
    return pl.pallas_call(
        _edge_b_pp_alias_body, grid=(nblk,), in_specs=[any_spec] + specs,
        out_specs=out_spec, out_shape=out_shape,
        input_output_aliases={0: 0})(prev, *args)


def _edge_b_lp_body(ev_ref, ea_ref, td_ref, tl_ref, e1, e2, be, wf1, wf2, bf,
                    sden, plm, pbm, pnv, r12, c12, out_ref):
    ev, ea, td, tl = ev_ref[...], ea_ref[...], td_ref[...], tl_ref[...]
    w = ev / (td @ sden[...] + 1e-9)
    new_lp = jnp.maximum(ea @ e1[...] + w @ e2[...] + be[...], 0.0)
    d12 = tl @ plm[...] - td @ pbm[...]
    n2 = (d12 * d12) @ c12[...]
    rn = 1.0 / jnp.sqrt(jnp.maximum(n2, 1e-8))
    t12 = (td @ pnv[...]) * (d12 * rn)
    out_ref[...] = new_lp @ wf1[...] + (t12 @ r12[...]) @ wf2[...] + bf[...]


def _edge_b_lp(ev, eattr, g2, n_out, W_eup, b_eup, W_flp, b_flp):
    epad = ev.shape[0]
    off = epad // _BLK_E
    consts = [jnp.asarray(m) for m in (_SDEN_LP, _PL, _PB, _PNV, _R12, _C12)]
    return pl.pallas_call(
        _edge_b_lp_body,
        grid=(pl.cdiv(n_out, _BLK_E),),
        in_specs=[_rows((_BLK_E, 128)), _rows((_BLK_E, 128)),
                  _rows((_BLK_E, 32)),
                  pl.BlockSpec((_BLK_E, 32), lambda i: (i + off, 0)),
                  _full((128, 128)), _full((128, 128)), _full((1, 128)),
                  _full((128, 128)), _full((4, 128)), _full((1, 128)),
                  _full((32, 128)), _full((32, 12)), _full((32, 12)),
                  _full((32, 12)), _full((12, 4)), _full((12, 1))],
        out_specs=_rows((_BLK_E, 128)),
        out_shape=jax.ShapeDtypeStruct((n_out, 128), jnp.float32),
    )(ev, eattr, g2, g2, W_eup[:128], W_eup[128:], b_eup.reshape(1, 128),
      W_flp[:128], W_flp[128:], b_flp.reshape(1, 128), *consts)


def _pad_rows(x, n):
    return jnp.pad(x, ((0, n - x.shape[0]),) + ((0, 0),) * (x.ndim - 1))


@jax.jit
def _run(prot_scalars, prot_vectors, lig_scalars, lig_vectors, pr_pr_eattr,
         lig_pr_eattr, pr_pr_edge_index, lig_pr_edge_index, lig_coords,
         backbone_coords, W_msg0, b_msg0, a_attn0, W_val0, W_head0, W_eup0,
         b_eup0, W_msg1, b_msg1, a_attn1, W_val1, W_head1, W_eup1, b_eup1,
         W_n1, b_n1, W_n2, b_n2, W_gate, W_flp, b_flp, W_fpp, b_fpp):
    prot = _pad_rows(prot_scalars, N_PAD)
    tab1 = jnp.concatenate([prot, lig_scalars], axis=0)
    ep, el = pr_pr_edge_index, lig_pr_edge_index
    pps = jnp.pad(ep[0], (0, EPP_PAD - E_PP))
    ppd = jnp.pad(ep[1], (0, EPP_PAD - E_PP))
    lps = jnp.pad(el[0], (0, ELP_PAD - E_LP))
    lpd = jnp.pad(el[1], (0, ELP_PAD - E_LP))
    ppd_sc = jnp.pad(ep[1], (0, EPP_PAD - E_PP), constant_values=TRASH)
    lpd_sc = jnp.pad(el[1], (0, ELP_PAD - E_LP), constant_values=TRASH)

    half = EPP_PAD // 2
    hb = half // _BLK_E
    g1ppA = _gather(tab1, jnp.concatenate([pps[:half], ppd[:half]]), 128)
    g1ppB = _gather(tab1, jnp.concatenate([pps[half:], ppd[half:]]), 128)
    g1lp = _gather(tab1, jnp.concatenate([lps + N_PAD, lpd]), 128)

    eattr_pp = _pad_rows(pr_pr_eattr, EPP_PAD)
    eattr_lp = _pad_rows(lig_pr_eattr, ELP_PAD)

    evA, exA = _edge_a(g1ppA, eattr_pp, W_msg0, b_msg0, a_attn0, W_val0,
                       epad=EPP_PAD)
    ev_pp, ex_pp = _edge_a(g1ppB, eattr_pp, W_msg0, b_msg0, a_attn0, W_val0,
                           epad=EPP_PAD, boff=hb, prev=(evA, exA))
    ev_lp, ex_lp = _edge_a(g1lp, eattr_lp, W_msg1, b_msg1, a_attn1, W_val1)

    zeros = jnp.zeros((N_PAD, 128), jnp.float32)
    zeros_den = jnp.zeros((N_PAD * 4,), jnp.float32)
    idx4_pp = (ppd_sc[:, None] * 4 + jnp.arange(4, dtype=jnp.int32)[None, :])
    idx4_lp = (lpd_sc[:, None] * 4 + jnp.arange(4, dtype=jnp.int32)[None, :])
    npp, dpp, nlp, dlp = _scatter(
        ev_pp, ex_pp.reshape(-1, CHUNK),
        ppd_sc.reshape(-1, CHUNK), idx4_pp.reshape(-1, CHUNK),
        ev_lp, ex_lp.reshape(-1, CHUNK),
        lpd_sc.reshape(-1, CHUNK), idx4_lp.reshape(-1, CHUNK),
        zeros, zeros_den)
    dpp = dpp.reshape(2, N_PAD, 4)
    dlp = dlp.reshape(2, N_PAD, 4)

    pvec16 = jnp.pad(_pad_rows(prot_vectors.reshape(N_PR, 12), N_PAD),
                     ((0, 0), (0, 4)))
    bb16 = jnp.pad(_pad_rows(backbone_coords[:, 1], N_PAD), ((0, 0), (0, 13)))
    Wg16 = jnp.pad(W_gate, ((0, 0), (0, 12)))
    new_scal, nvec16, geom = _node(
        prot, pvec16, bb16, npp[:N_PAD], npp[N_PAD:], nlp[:N_PAD], nlp[N_PAD:],
        dpp[0], dpp[1], dlp[0], dlp[1],
        W_n1, b_n1, W_n2, b_n2, Wg16, W_head0, W_head1)

    ligtab = jnp.pad(lig_coords, ((0, 0), (0, 29)))
    tab2 = jnp.concatenate([geom, ligtab], axis=0)
    g2ppA = _gather(tab2, jnp.concatenate([pps[:half], ppd[:half]]), 32)
    g2ppB = _gather(tab2, jnp.concatenate([pps[half:], ppd[half:]]), 32)
    g2lp = _gather(tab2, jnp.concatenate([lpd, lps + N_PAD]), 32)

    ppoA = _edge_b_pp(ev_pp, eattr_pp, g2ppA, E_PP,
                      W_eup0, b_eup0, W_fpp, b_fpp, nblk=hb)
    pp_out = _edge_b_pp(ev_pp, eattr_pp, g2ppB, E_PP,
                        W_eup0, b_eup0, W_fpp, b_fpp,
                        nblk=pl.cdiv(E_PP, _BLK_E) - hb, boff=hb, prev=ppoA)
    lp_out = _edge_b_lp(ev_lp, eattr_lp, g2lp, E_LP,
                        W_eup1, b_eup1, W_flp, b_flp)

    return (new_scal,
            nvec16[:, :12].reshape(N_PR, NV, 3),
            pp_out,
            lp_out)


def kernel(*args):
    return _run(*args)
